# Initial kernel scaffold; baseline (speedup 1.0000x reference)
import jax
import jax.numpy as jnp
from jax.experimental import pallas as pl


def kernel(feats, edge_index, W1, b1, codebook, Wd1, bd1, Wd2, bd2, W2, b2, Wl, bl):
    raise NotImplementedError("write your pallas kernel here")


# speedup vs baseline: 1.7340x; 1.7340x over previous
import functools

import jax
import jax.numpy as jnp
from jax import lax
from jax.experimental import pallas as pl
from jax.experimental.pallas import tpu as pltpu
from jax.experimental.pallas import tpu_sc as plsc

N = 4096
E = 65536
D = 128
KC = 8192
DOUT = 64

NW = 32
EP = E // NW
EPC = E // 16
CH_ROWS = 128
CH = CH_ROWS * N
NCH = N // CH_ROWS
STRIPE = CH // 16
BI = 256
KJ = 1024


def _adj_build_body(src_hbm, dst_hbm, adj_hbm, deg_hbm,
                    src_v, dst_v, key_g, pos_b, idx_b, val_b, ones_b, zbuf,
                    din_s, dout_s, chunk_s):
    cid = lax.axis_index("c")
    sid = lax.axis_index("s")
    eoff = sid * EPC

    pltpu.sync_copy(src_hbm.at[pl.ds(eoff, EPC)], src_v)
    pltpu.sync_copy(dst_hbm.at[pl.ds(eoff, EPC)], dst_v)

    def _zero(i, _):
        zbuf[pl.ds(i * 16, 16)] = jnp.zeros((16,), jnp.float32)
        return 0
    lax.fori_loop(0, STRIPE // 16, _zero, 0)

    def _prep(i, _):
        j = i // 8
        c = i % 8
        off = j * 128 + c * 16
        sv = src_v[pl.ds(off, 16)]
        dv = dst_v[pl.ds(off, 16)]
        key_g[j, pl.ds(c * 16, 16)] = dv * N + sv
        pos_b[j, pl.ds(c * 16, 16)] = (
            lax.iota(jnp.int32, 16) + (CH + off))
        ones_b[j, pl.ds(c * 16, 16)] = jnp.ones((16,), jnp.float32)
        return 0
    lax.fori_loop(0, (EPC // 128) * 8, _prep, 0)

    @pl.when(sid == 0)
    def _():
        pltpu.sync_copy(zbuf.at[pl.ds(0, N)], din_s)
        pltpu.sync_copy(zbuf.at[pl.ds(0, N)], dout_s)
    plsc.subcore_barrier()

    def _deg_row(jj, _):
        j = jj + cid * 16
        def _mk_d(c, _2):
            off = j * 128 + c * 16
            idx_b[j, pl.ds(c * 16, 16)] = dst_v[pl.ds(off, 16)]
            return 0
        lax.fori_loop(0, 8, _mk_d, 0)
        pltpu.sync_copy(ones_b.at[j], din_s.at[idx_b.at[j]], add=True)
        def _mk_s(c, _2):
            off = j * 128 + c * 16
            idx_b[j, pl.ds(c * 16, 16)] = src_v[pl.ds(off, 16)]
            return 0
        lax.fori_loop(0, 8, _mk_s, 0)
        pltpu.sync_copy(ones_b.at[j], dout_s.at[idx_b.at[j]], add=True)
        return 0
    lax.fori_loop(0, 16, _deg_row, 0)

    plsc.subcore_barrier()
    @pl.when(sid == 0)
    def _():
        pltpu.sync_copy(din_s, deg_hbm.at[cid, 0])
        pltpu.sync_copy(dout_s, deg_hbm.at[cid, 1])

    for ch in range(NCH // 2):
        chunk_id = ch * 2 + cid
        base = chunk_id * CH
        pltpu.sync_copy(zbuf, chunk_s.at[pl.ds(sid * STRIPE, STRIPE)])
        plsc.subcore_barrier()

        def _scat_row(j, _):
            def _mk(c, _2):
                off = j * 128 + c * 16
                k16 = key_g[j, pl.ds(c * 16, 16)] - base
                m = (k16 >= 0) & (k16 < CH)
                idx_b[j, pl.ds(c * 16, 16)] = jnp.where(
                    m, k16, pos_b[j, pl.ds(c * 16, 16)])
                val_b[j, pl.ds(c * 16, 16)] = jnp.where(m, 1.0, 0.0)
                return 0
            lax.fori_loop(0, 8, _mk, 0)
            pltpu.sync_copy(val_b.at[j], chunk_s.at[idx_b.at[j]], add=True)
            return 0
        lax.fori_loop(0, EPC // 128, _scat_row, 0)

        plsc.subcore_barrier()
        pltpu.sync_copy(chunk_s.at[pl.ds(sid * STRIPE, STRIPE)],
                        adj_hbm.at[pl.ds(base + sid * STRIPE, STRIPE)])
        plsc.subcore_barrier()


@functools.cache
def _get_adj_build():
    return functools.partial(
        pl.kernel,
        out_type=[jax.ShapeDtypeStruct((N * N,), jnp.float32),
                  jax.ShapeDtypeStruct((2, 2, N), jnp.float32)],
        mesh=plsc.VectorSubcoreMesh(core_axis_name="c", subcore_axis_name="s"),
        scratch_types=[
        pltpu.VMEM((EPC,), jnp.int32),
        pltpu.VMEM((EPC,), jnp.int32),
        pltpu.VMEM((EPC // 128, 128), jnp.int32),
        pltpu.VMEM((EPC // 128, 128), jnp.int32),
        pltpu.VMEM((EPC // 128, 128), jnp.int32),
        pltpu.VMEM((EPC // 128, 128), jnp.float32),
        pltpu.VMEM((EPC // 128, 128), jnp.float32),
        pltpu.VMEM((STRIPE,), jnp.float32),
        pltpu.VMEM_SHARED((N,), jnp.float32),
        pltpu.VMEM_SHARED((N,), jnp.float32),
        pltpu.VMEM_SHARED((CH + EPC,), jnp.float32),
        ],
    )(_adj_build_body)


BPW = N // NW


def _quant_gather_body(cn_hbm, ind_hbm, out_hbm, idx_v, rows_v, sem):
    cid = lax.axis_index("c")
    sid = lax.axis_index("s")
    wid = sid * 2 + cid
    base = wid * BPW
    pltpu.sync_copy(ind_hbm.at[pl.ds(base, BPW)], idx_v)
    pltpu.async_copy(cn_hbm.at[idx_v], rows_v, sem).wait()
    pltpu.sync_copy(rows_v, out_hbm.at[pl.ds(base, BPW)])


@functools.cache
def _get_quant_gather():
    return functools.partial(
        pl.kernel,
        out_type=jax.ShapeDtypeStruct((N, D), jnp.float32),
        mesh=plsc.VectorSubcoreMesh(core_axis_name="c", subcore_axis_name="s"),
        scratch_types=[
            pltpu.VMEM((BPW,), jnp.int32),
            pltpu.VMEM((BPW, D), jnp.float32),
            pltpu.SemaphoreType.DMA,
        ],
    )(_quant_gather_body)


def _exact_agg(a, xs):
    f32 = jnp.float32
    bf = jnp.bfloat16
    ah = a.astype(bf)
    al = (a - ah.astype(f32)).astype(bf)
    x1 = xs.astype(bf)
    r1 = xs - x1.astype(f32)
    x2 = r1.astype(bf)
    x3 = (r1 - x2.astype(f32)).astype(bf)
    dot = lambda p, q: jax.lax.dot_general(
        p, q, (((1,), (0,)), ((), ())), preferred_element_type=f32)
    return ((dot(ah, x1) + dot(ah, x2) + dot(ah, x3)) +
            (dot(al, x1) + dot(al, x2) + dot(al, x3)))


def _gcn_body(adj_ref, x_ref, degp_ref, w_ref, b_ref, o_ref):
    i = pl.program_id(0)
    deg_out = degp_ref[0, 1, :] + degp_ref[1, 1, :]
    nsrc = lax.rsqrt(jnp.clip(deg_out, 1.0, None))
    deg_in = (degp_ref[0, 0, pl.ds(i * BI, BI)] +
              degp_ref[1, 0, pl.ds(i * BI, BI)])
    ndst = lax.rsqrt(jnp.clip(deg_in, 1.0, None))
    xs = x_ref[...] * nsrc[:, None]
    agg = _exact_agg(adj_ref[...], xs)
    agg = agg * ndst[:, None]
    h = jnp.dot(agg, w_ref[...], preferred_element_type=jnp.float32) + b_ref[...]
    o_ref[...] = jnp.maximum(h, 0.0)


def _gcn_layer_pallas(adj, x, degp, w, b):
    return pl.pallas_call(
        _gcn_body,
        grid=(N // BI,),
        in_specs=[
            pl.BlockSpec((BI, N), lambda i: (i, 0)),
            pl.BlockSpec((N, D), lambda i: (0, 0)),
            pl.BlockSpec((2, 2, N), lambda i: (0, 0, 0)),
            pl.BlockSpec((D, D), lambda i: (0, 0)),
            pl.BlockSpec((1, D), lambda i: (0, 0)),
        ],
        out_specs=pl.BlockSpec((BI, D), lambda i: (i, 0)),
        out_shape=jax.ShapeDtypeStruct((N, D), jnp.float32),
    )(adj, x, degp, w, b)


def _l2_body(x_ref, o_ref):
    x = x_ref[...]
    n = jnp.sqrt(jnp.sum(x * x, axis=1, keepdims=True))
    o_ref[...] = x / jnp.clip(n, 1e-12, None)


def _l2norm_pallas(x):
    m = x.shape[0]
    return pl.pallas_call(
        _l2_body,
        grid=(m // 1024,),
        in_specs=[pl.BlockSpec((1024, D), lambda i: (i, 0))],
        out_specs=pl.BlockSpec((1024, D), lambda i: (i, 0)),
        out_shape=jax.ShapeDtypeStruct((m, D), jnp.float32),
    )(x)


def _vq_body(h1_ref, cn_ref, dist_ref, ind_ref, rmax_ref, ridx_ref):
    j = pl.program_id(1)
    h = h1_ref[...]
    n = jnp.sqrt(jnp.sum(h * h, axis=1, keepdims=True))
    hn = h / jnp.clip(n, 1e-12, None)
    d = lax.dot_general(hn, cn_ref[...], (((1,), (1,)), ((), ())),
                        preferred_element_type=jnp.float32)
    dist_ref[...] = d
    tmax = jnp.max(d, axis=1, keepdims=True)
    col = lax.broadcasted_iota(jnp.int32, (BI, KJ), 1)
    tidx = jnp.min(jnp.where(d >= tmax, col, jnp.int32(2 ** 30)),
                   axis=1, keepdims=True) + j * KJ

    @pl.when(j == 0)
    def _():
        rmax_ref[...] = tmax
        ridx_ref[...] = tidx

    @pl.when(j > 0)
    def _():
        better = tmax > rmax_ref[...]
        ridx_ref[...] = jnp.where(better, tidx, ridx_ref[...])
        rmax_ref[...] = jnp.maximum(rmax_ref[...], tmax)

    @pl.when(j == KC // KJ - 1)
    def _():
        ind_ref[...] = ridx_ref[...]


def _vq_pallas(h1, cn):
    return pl.pallas_call(
        _vq_body,
        grid=(N // BI, KC // KJ),
        in_specs=[
            pl.BlockSpec((BI, D), lambda i, j: (i, 0)),
            pl.BlockSpec((KJ, D), lambda i, j: (j, 0)),
        ],
        out_specs=[
            pl.BlockSpec((BI, KJ), lambda i, j: (i, j)),
            pl.BlockSpec((BI, 1), lambda i, j: (i, 0)),
        ],
        out_shape=[
            jax.ShapeDtypeStruct((N, KC), jnp.float32),
            jax.ShapeDtypeStruct((N, 1), jnp.int32),
        ],
        scratch_shapes=[
            pltpu.VMEM((BI, 1), jnp.float32),
            pltpu.VMEM((BI, 1), jnp.int32),
        ],
    )(h1, cn)


def _dec_body(q_ref, h_ref, wd1_ref, bd1_ref, wd2_ref, bd2_ref,
              qe_ref, c_ref, f_ref):
    q = q_ref[...]
    h = h_ref[...]
    qe = lax.dot_general(q, wd1_ref[...], (((1,), (1,)), ((), ())),
                         preferred_element_type=jnp.float32) + bd1_ref[...]
    qn = lax.dot_general(q, wd2_ref[...], (((1,), (1,)), ((), ())),
                         preferred_element_type=jnp.float32) + bd2_ref[...]
    qe_ref[...] = qe
    c_ref[0, 0] = jnp.sum((q - h) ** 2)
    f_ref[0, 0] = jnp.sum((h - qn) ** 2)


def _dec_pallas(quant, h1, wd1, bd1, wd2, bd2):
    return pl.pallas_call(
        _dec_body,
        out_specs=[
            pl.BlockSpec(memory_space=pltpu.VMEM),
            pl.BlockSpec(memory_space=pltpu.SMEM),
            pl.BlockSpec(memory_space=pltpu.SMEM),
        ],
        out_shape=[
            jax.ShapeDtypeStruct((N, D), jnp.float32),
            jax.ShapeDtypeStruct((1, 1), jnp.float32),
            jax.ShapeDtypeStruct((1, 1), jnp.float32),
        ],
    )(quant, h1, wd1, bd1, wd2, bd2)


def _adjq_body(qei_ref, qej_ref, adj_ref, out_ref):
    i = pl.program_id(0)
    j = pl.program_id(1)
    t = lax.dot_general(qei_ref[...], qej_ref[...], (((1,), (1,)), ((), ())),
                        preferred_element_type=jnp.float32)
    a = adj_ref[...]
    tmn = jnp.min(t)
    tmx = jnp.max(t)
    s1 = jnp.sum(t)
    s2 = jnp.sum(t * t)
    c = jnp.sum(a * t)
    a2 = jnp.sum(a * a)
    a1 = jnp.sum(a)
    first = jnp.logical_and(i == 0, j == 0)

    @pl.when(first)
    def _():
        out_ref[0] = tmn
        out_ref[1] = tmx
        out_ref[2] = s1
        out_ref[3] = s2
        out_ref[4] = c
        out_ref[5] = a2
        out_ref[6] = a1
        out_ref[7] = 0.0

    @pl.when(jnp.logical_not(first))
    def _():
        out_ref[0] = jnp.minimum(out_ref[0], tmn)
        out_ref[1] = jnp.maximum(out_ref[1], tmx)
        out_ref[2] = out_ref[2] + s1
        out_ref[3] = out_ref[3] + s2
        out_ref[4] = out_ref[4] + c
        out_ref[5] = out_ref[5] + a2
        out_ref[6] = out_ref[6] + a1


def _adjq_pallas(q_edge, adj):
    return pl.pallas_call(
        _adjq_body,
        grid=(N // BI, N // BI),
        in_specs=[
            pl.BlockSpec((BI, D), lambda i, j: (i, 0)),
            pl.BlockSpec((BI, D), lambda i, j: (j, 0)),
            pl.BlockSpec((BI, BI), lambda i, j: (i, j)),
        ],
        out_specs=pl.BlockSpec(memory_space=pltpu.SMEM),
        out_shape=jax.ShapeDtypeStruct((8,), jnp.float32),
    )(q_edge, q_edge, adj)


def _gcn2_body(adj_ref, x_ref, degp_ref, w_ref, b_ref, wl_ref, bl_ref,
               h2_ref, o_ref):
    i = pl.program_id(0)
    deg_out = degp_ref[0, 1, :] + degp_ref[1, 1, :]
    nsrc = lax.rsqrt(jnp.clip(deg_out, 1.0, None))
    deg_in = (degp_ref[0, 0, pl.ds(i * BI, BI)] +
              degp_ref[1, 0, pl.ds(i * BI, BI)])
    ndst = lax.rsqrt(jnp.clip(deg_in, 1.0, None))
    xs = x_ref[...] * nsrc[:, None]
    agg = _exact_agg(adj_ref[...], xs)
    agg = agg * ndst[:, None]
    h = jnp.dot(agg, w_ref[...], preferred_element_type=jnp.float32) + b_ref[...]
    h2 = jnp.maximum(h, 0.0)
    h2_ref[...] = h2
    o_ref[...] = lax.dot_general(h2, wl_ref[...], (((1,), (1,)), ((), ())),
                                 preferred_element_type=jnp.float32) + bl_ref[...]


def _gcn2_pallas(adj, q_edge, degp, w2, b2, wl, bl):
    return pl.pallas_call(
        _gcn2_body,
        grid=(N // BI,),
        in_specs=[
            pl.BlockSpec((BI, N), lambda i: (i, 0)),
            pl.BlockSpec((N, D), lambda i: (0, 0)),
            pl.BlockSpec((2, 2, N), lambda i: (0, 0, 0)),
            pl.BlockSpec((D, D), lambda i: (0, 0)),
            pl.BlockSpec((1, D), lambda i: (0, 0)),
            pl.BlockSpec((DOUT, D), lambda i: (0, 0)),
            pl.BlockSpec((1, DOUT), lambda i: (0, 0)),
        ],
        out_specs=[
            pl.BlockSpec((BI, D), lambda i: (i, 0)),
            pl.BlockSpec((BI, DOUT), lambda i: (i, 0)),
        ],
        out_shape=[
            jax.ShapeDtypeStruct((N, D), jnp.float32),
            jax.ShapeDtypeStruct((N, DOUT), jnp.float32),
        ],
    )(adj, q_edge, degp, w2, b2, wl, bl)


def kernel(feats, edge_index, W1, b1, codebook, Wd1, bd1, Wd2, bd2,
           W2, b2, Wl, bl):
    src = edge_index[0]
    dst = edge_index[1]

    adj_flat, degp = _get_adj_build()(src, dst)
    adj = adj_flat.reshape(N, N)

    h1 = _gcn_layer_pallas(adj, feats, degp, W1, b1.reshape(1, D))
    cn = _l2norm_pallas(codebook)
    dist, ind2 = _vq_pallas(h1, cn)
    ind = ind2.reshape(N)
    quant = _get_quant_gather()(cn, ind)

    q_edge, sse_commit, sse_node = _dec_pallas(
        quant, h1, Wd1, bd1.reshape(1, D), Wd2, bd2.reshape(1, D))

    stats = _adjq_pallas(q_edge, adj)
    mn, mx, s1, s2, c, a2, a1 = (stats[0], stats[1], stats[2], stats[3],
                                 stats[4], stats[5], stats[6])

    h2, out = _gcn2_pallas(adj, q_edge, degp, W2, b2.reshape(1, D),
                           Wl, bl.reshape(1, DOUT))

    nn = jnp.float32(N) * jnp.float32(N)
    den = mx - mn
    s2n = (s2 - 2.0 * mn * s1 + nn * mn * mn) / (den * den)
    cxn = (c - mn * a1) / den
    edge_rec = jnp.sqrt((a2 - 2.0 * cxn + s2n) / nn)
    feature_rec = sse_node[0, 0] / jnp.float32(N * D)
    commit = 0.25 * sse_commit[0, 0] / jnp.float32(N * D)
    loss = feature_rec + edge_rec + commit

    return (h1, quant, h2, out, loss, dist, cn)


# speedup vs baseline: 1.9158x; 1.1048x over previous
import functools

import jax
import jax.numpy as jnp
from jax import lax
from jax.experimental import pallas as pl
from jax.experimental.pallas import tpu as pltpu
from jax.experimental.pallas import tpu_sc as plsc

N = 4096
E = 65536
D = 128
KC = 8192
DOUT = 64

NW = 32
EP = E // NW
EPC = E // 16
CH_ROWS = 256
CH = CH_ROWS * N
NCH = N // CH_ROWS
STRIPE = CH // 16
ZB = 4096
BI = 256
KJ = 1024


def _adj_build_body(src_hbm, dst_hbm, adj_hbm, deg_hbm,
                    src_v, dst_v, key_g, pos_b, idx_b, val_b, ones_b, zbuf,
                    din_s, dout_s, chunk_s):
    cid = lax.axis_index("c")
    sid = lax.axis_index("s")
    eoff = sid * EPC

    pltpu.sync_copy(src_hbm.at[pl.ds(eoff, EPC)], src_v)
    pltpu.sync_copy(dst_hbm.at[pl.ds(eoff, EPC)], dst_v)

    def _zero(i, _):
        zbuf[pl.ds(i * 16, 16)] = jnp.zeros((16,), jnp.float32)
        return 0
    lax.fori_loop(0, ZB // 16, _zero, 0)

    def _prep(i, _):
        j = i // 8
        c = i % 8
        off = j * 128 + c * 16
        sv = src_v[pl.ds(off, 16)]
        dv = dst_v[pl.ds(off, 16)]
        key_g[j, pl.ds(c * 16, 16)] = dv * N + sv
        pos_b[j, pl.ds(c * 16, 16)] = (
            lax.iota(jnp.int32, 16) + (CH + off))
        ones_b[j, pl.ds(c * 16, 16)] = jnp.ones((16,), jnp.float32)
        return 0
    lax.fori_loop(0, (EPC // 128) * 8, _prep, 0)

    @pl.when(sid == 0)
    def _():
        pltpu.sync_copy(zbuf.at[pl.ds(0, N)], din_s)
        pltpu.sync_copy(zbuf.at[pl.ds(0, N)], dout_s)
    plsc.subcore_barrier()

    def _deg_row(jj, _):
        j = jj + cid * 16
        def _mk_d(c, _2):
            off = j * 128 + c * 16
            idx_b[j, pl.ds(c * 16, 16)] = dst_v[pl.ds(off, 16)]
            return 0
        lax.fori_loop(0, 8, _mk_d, 0)
        pltpu.sync_copy(ones_b.at[j], din_s.at[idx_b.at[j]], add=True)
        def _mk_s(c, _2):
            off = j * 128 + c * 16
            idx_b[j, pl.ds(c * 16, 16)] = src_v[pl.ds(off, 16)]
            return 0
        lax.fori_loop(0, 8, _mk_s, 0)
        pltpu.sync_copy(ones_b.at[j], dout_s.at[idx_b.at[j]], add=True)
        return 0
    lax.fori_loop(0, 16, _deg_row, 0)

    plsc.subcore_barrier()
    @pl.when(sid == 0)
    def _():
        pltpu.sync_copy(din_s, deg_hbm.at[cid, 0])
        pltpu.sync_copy(dout_s, deg_hbm.at[cid, 1])

    for ch in range(NCH // 2):
        chunk_id = ch * 2 + cid
        base = chunk_id * CH

        def _zstripe(k, _):
            pltpu.sync_copy(
                zbuf, chunk_s.at[pl.ds(sid * STRIPE + k * ZB, ZB)])
            return 0
        lax.fori_loop(0, STRIPE // ZB, _zstripe, 0)
        plsc.subcore_barrier()

        def _scat_row(j, _):
            def _mk(c, _2):
                off = j * 128 + c * 16
                k16 = key_g[j, pl.ds(c * 16, 16)] - base
                m = (k16 >= 0) & (k16 < CH)
                idx_b[j, pl.ds(c * 16, 16)] = jnp.where(
                    m, k16, pos_b[j, pl.ds(c * 16, 16)])
                val_b[j, pl.ds(c * 16, 16)] = jnp.where(m, 1.0, 0.0)
                return 0
            lax.fori_loop(0, 8, _mk, 0)
            pltpu.sync_copy(val_b.at[j], chunk_s.at[idx_b.at[j]], add=True)
            return 0
        lax.fori_loop(0, EPC // 128, _scat_row, 0)

        plsc.subcore_barrier()
        pltpu.sync_copy(chunk_s.at[pl.ds(sid * STRIPE, STRIPE)],
                        adj_hbm.at[pl.ds(base + sid * STRIPE, STRIPE)])
        plsc.subcore_barrier()


@functools.cache
def _get_adj_build():
    return functools.partial(
        pl.kernel,
        out_type=[jax.ShapeDtypeStruct((N * N,), jnp.float32),
                  jax.ShapeDtypeStruct((2, 2, N), jnp.float32)],
        mesh=plsc.VectorSubcoreMesh(core_axis_name="c", subcore_axis_name="s"),
        scratch_types=[
        pltpu.VMEM((EPC,), jnp.int32),
        pltpu.VMEM((EPC,), jnp.int32),
        pltpu.VMEM((EPC // 128, 128), jnp.int32),
        pltpu.VMEM((EPC // 128, 128), jnp.int32),
        pltpu.VMEM((EPC // 128, 128), jnp.int32),
        pltpu.VMEM((EPC // 128, 128), jnp.float32),
        pltpu.VMEM((EPC // 128, 128), jnp.float32),
        pltpu.VMEM((ZB,), jnp.float32),
        pltpu.VMEM_SHARED((N,), jnp.float32),
        pltpu.VMEM_SHARED((N,), jnp.float32),
        pltpu.VMEM_SHARED((CH + EPC,), jnp.float32),
        ],
    )(_adj_build_body)


BPW = N // NW


def _quant_gather_body(cn_hbm, ind_hbm, out_hbm, idx_v, rows_v, sem):
    cid = lax.axis_index("c")
    sid = lax.axis_index("s")
    wid = sid * 2 + cid
    base = wid * BPW
    pltpu.sync_copy(ind_hbm.at[pl.ds(base, BPW)], idx_v)
    pltpu.async_copy(cn_hbm.at[idx_v], rows_v, sem).wait()
    pltpu.sync_copy(rows_v, out_hbm.at[pl.ds(base, BPW)])


@functools.cache
def _get_quant_gather():
    return functools.partial(
        pl.kernel,
        out_type=jax.ShapeDtypeStruct((N, D), jnp.float32),
        mesh=plsc.VectorSubcoreMesh(core_axis_name="c", subcore_axis_name="s"),
        scratch_types=[
            pltpu.VMEM((BPW,), jnp.int32),
            pltpu.VMEM((BPW, D), jnp.float32),
            pltpu.SemaphoreType.DMA,
        ],
    )(_quant_gather_body)


def _exact_agg(a, xs):
    f32 = jnp.float32
    bf = jnp.bfloat16
    ah = a.astype(bf)
    al = (a - ah.astype(f32)).astype(bf)
    x1 = xs.astype(bf)
    r1 = xs - x1.astype(f32)
    x2 = r1.astype(bf)
    x3 = (r1 - x2.astype(f32)).astype(bf)
    dot = lambda p, q: jax.lax.dot_general(
        p, q, (((1,), (0,)), ((), ())), preferred_element_type=f32)
    return (dot(ah, x1) + dot(ah, x2)) + (dot(ah, x3) + dot(al, x1))


def _gcn_body(adj_ref, x_ref, degp_ref, w_ref, b_ref, o_ref):
    i = pl.program_id(0)
    deg_out = degp_ref[0, 1, :] + degp_ref[1, 1, :]
    nsrc = lax.rsqrt(jnp.clip(deg_out, 1.0, None))
    deg_in = (degp_ref[0, 0, pl.ds(i * BI, BI)] +
              degp_ref[1, 0, pl.ds(i * BI, BI)])
    ndst = lax.rsqrt(jnp.clip(deg_in, 1.0, None))
    xs = x_ref[...] * nsrc[:, None]
    agg = _exact_agg(adj_ref[...], xs)
    agg = agg * ndst[:, None]
    h = jnp.dot(agg, w_ref[...], preferred_element_type=jnp.float32) + b_ref[...]
    o_ref[...] = jnp.maximum(h, 0.0)


def _gcn_layer_pallas(adj, x, degp, w, b):
    return pl.pallas_call(
        _gcn_body,
        grid=(N // BI,),
        in_specs=[
            pl.BlockSpec((BI, N), lambda i: (i, 0)),
            pl.BlockSpec((N, D), lambda i: (0, 0)),
            pl.BlockSpec((2, 2, N), lambda i: (0, 0, 0)),
            pl.BlockSpec((D, D), lambda i: (0, 0)),
            pl.BlockSpec((1, D), lambda i: (0, 0)),
        ],
        out_specs=pl.BlockSpec((BI, D), lambda i: (i, 0)),
        out_shape=jax.ShapeDtypeStruct((N, D), jnp.float32),
    )(adj, x, degp, w, b)


def _l2_body(x_ref, o_ref):
    x = x_ref[...]
    n = jnp.sqrt(jnp.sum(x * x, axis=1, keepdims=True))
    o_ref[...] = x / jnp.clip(n, 1e-12, None)


def _l2norm_pallas(x):
    m = x.shape[0]
    return pl.pallas_call(
        _l2_body,
        grid=(m // 1024,),
        in_specs=[pl.BlockSpec((1024, D), lambda i: (i, 0))],
        out_specs=pl.BlockSpec((1024, D), lambda i: (i, 0)),
        out_shape=jax.ShapeDtypeStruct((m, D), jnp.float32),
    )(x)


def _vq_body(h1_ref, cn_ref, dist_ref, ind_ref, rmax_ref, ridx_ref, hn_ref):
    j = pl.program_id(1)

    @pl.when(j == 0)
    def _():
        h = h1_ref[...]
        n = jnp.sqrt(jnp.sum(h * h, axis=1, keepdims=True))
        hn_ref[...] = h / jnp.clip(n, 1e-12, None)

    d = lax.dot_general(hn_ref[...], cn_ref[...], (((1,), (1,)), ((), ())),
                        preferred_element_type=jnp.float32)
    dist_ref[...] = d
    tmax = jnp.max(d, axis=1, keepdims=True)
    col = lax.broadcasted_iota(jnp.int32, (BI, KJ), 1)
    tidx = jnp.min(jnp.where(d >= tmax, col, jnp.int32(2 ** 30)),
                   axis=1, keepdims=True) + j * KJ

    @pl.when(j == 0)
    def _():
        rmax_ref[...] = tmax
        ridx_ref[...] = tidx

    @pl.when(j > 0)
    def _():
        better = tmax > rmax_ref[...]
        ridx_ref[...] = jnp.where(better, tidx, ridx_ref[...])
        rmax_ref[...] = jnp.maximum(rmax_ref[...], tmax)

    @pl.when(j == KC // KJ - 1)
    def _():
        ind_ref[...] = ridx_ref[...]


def _vq_pallas(h1, cn):
    return pl.pallas_call(
        _vq_body,
        grid=(N // BI, KC // KJ),
        in_specs=[
            pl.BlockSpec((BI, D), lambda i, j: (i, 0)),
            pl.BlockSpec((KJ, D), lambda i, j: (j, 0)),
        ],
        out_specs=[
            pl.BlockSpec((BI, KJ), lambda i, j: (i, j)),
            pl.BlockSpec((BI, 1), lambda i, j: (i, 0)),
        ],
        out_shape=[
            jax.ShapeDtypeStruct((N, KC), jnp.float32),
            jax.ShapeDtypeStruct((N, 1), jnp.int32),
        ],
        scratch_shapes=[
            pltpu.VMEM((BI, 1), jnp.float32),
            pltpu.VMEM((BI, 1), jnp.int32),
            pltpu.VMEM((BI, D), jnp.float32),
        ],
    )(h1, cn)


def _dec_body(q_ref, h_ref, wd1_ref, bd1_ref, wd2_ref, bd2_ref,
              qe_ref, c_ref, f_ref):
    q = q_ref[...]
    h = h_ref[...]
    qe = lax.dot_general(q, wd1_ref[...], (((1,), (1,)), ((), ())),
                         preferred_element_type=jnp.float32) + bd1_ref[...]
    qn = lax.dot_general(q, wd2_ref[...], (((1,), (1,)), ((), ())),
                         preferred_element_type=jnp.float32) + bd2_ref[...]
    qe_ref[...] = qe
    c_ref[0, 0] = jnp.sum((q - h) ** 2)
    f_ref[0, 0] = jnp.sum((h - qn) ** 2)


def _dec_pallas(quant, h1, wd1, bd1, wd2, bd2):
    return pl.pallas_call(
        _dec_body,
        out_specs=[
            pl.BlockSpec(memory_space=pltpu.VMEM),
            pl.BlockSpec(memory_space=pltpu.SMEM),
            pl.BlockSpec(memory_space=pltpu.SMEM),
        ],
        out_shape=[
            jax.ShapeDtypeStruct((N, D), jnp.float32),
            jax.ShapeDtypeStruct((1, 1), jnp.float32),
            jax.ShapeDtypeStruct((1, 1), jnp.float32),
        ],
    )(quant, h1, wd1, bd1, wd2, bd2)


def _adjq_body(qei_ref, qej_ref, adj_ref, out_ref):
    i = pl.program_id(0)
    j = pl.program_id(1)
    t = lax.dot_general(qei_ref[...], qej_ref[...], (((1,), (1,)), ((), ())),
                        preferred_element_type=jnp.float32)
    a = adj_ref[...]
    tmn = jnp.min(t)
    tmx = jnp.max(t)
    s1 = jnp.sum(t)
    s2 = jnp.sum(t * t)
    c = jnp.sum(a * t)
    a2 = jnp.sum(a * a)
    a1 = jnp.sum(a)
    first = jnp.logical_and(i == 0, j == 0)

    @pl.when(first)
    def _():
        out_ref[0] = tmn
        out_ref[1] = tmx
        out_ref[2] = s1
        out_ref[3] = s2
        out_ref[4] = c
        out_ref[5] = a2
        out_ref[6] = a1
        out_ref[7] = 0.0

    @pl.when(jnp.logical_not(first))
    def _():
        out_ref[0] = jnp.minimum(out_ref[0], tmn)
        out_ref[1] = jnp.maximum(out_ref[1], tmx)
        out_ref[2] = out_ref[2] + s1
        out_ref[3] = out_ref[3] + s2
        out_ref[4] = out_ref[4] + c
        out_ref[5] = out_ref[5] + a2
        out_ref[6] = out_ref[6] + a1


def _adjq_pallas(q_edge, adj):
    return pl.pallas_call(
        _adjq_body,
        grid=(N // BI, N // BI),
        in_specs=[
            pl.BlockSpec((BI, D), lambda i, j: (i, 0)),
            pl.BlockSpec((BI, D), lambda i, j: (j, 0)),
            pl.BlockSpec((BI, BI), lambda i, j: (i, j)),
        ],
        out_specs=pl.BlockSpec(memory_space=pltpu.SMEM),
        out_shape=jax.ShapeDtypeStruct((8,), jnp.float32),
    )(q_edge, q_edge, adj)


def _gcn2_body(adj_ref, x_ref, degp_ref, w_ref, b_ref, wl_ref, bl_ref,
               h2_ref, o_ref):
    i = pl.program_id(0)
    deg_out = degp_ref[0, 1, :] + degp_ref[1, 1, :]
    nsrc = lax.rsqrt(jnp.clip(deg_out, 1.0, None))
    deg_in = (degp_ref[0, 0, pl.ds(i * BI, BI)] +
              degp_ref[1, 0, pl.ds(i * BI, BI)])
    ndst = lax.rsqrt(jnp.clip(deg_in, 1.0, None))
    xs = x_ref[...] * nsrc[:, None]
    agg = _exact_agg(adj_ref[...], xs)
    agg = agg * ndst[:, None]
    h = jnp.dot(agg, w_ref[...], preferred_element_type=jnp.float32) + b_ref[...]
    h2 = jnp.maximum(h, 0.0)
    h2_ref[...] = h2
    o_ref[...] = lax.dot_general(h2, wl_ref[...], (((1,), (1,)), ((), ())),
                                 preferred_element_type=jnp.float32) + bl_ref[...]


def _gcn2_pallas(adj, q_edge, degp, w2, b2, wl, bl):
    return pl.pallas_call(
        _gcn2_body,
        grid=(N // BI,),
        in_specs=[
            pl.BlockSpec((BI, N), lambda i: (i, 0)),
            pl.BlockSpec((N, D), lambda i: (0, 0)),
            pl.BlockSpec((2, 2, N), lambda i: (0, 0, 0)),
            pl.BlockSpec((D, D), lambda i: (0, 0)),
            pl.BlockSpec((1, D), lambda i: (0, 0)),
            pl.BlockSpec((DOUT, D), lambda i: (0, 0)),
            pl.BlockSpec((1, DOUT), lambda i: (0, 0)),
        ],
        out_specs=[
            pl.BlockSpec((BI, D), lambda i: (i, 0)),
            pl.BlockSpec((BI, DOUT), lambda i: (i, 0)),
        ],
        out_shape=[
            jax.ShapeDtypeStruct((N, D), jnp.float32),
            jax.ShapeDtypeStruct((N, DOUT), jnp.float32),
        ],
    )(adj, q_edge, degp, w2, b2, wl, bl)


def kernel(feats, edge_index, W1, b1, codebook, Wd1, bd1, Wd2, bd2,
           W2, b2, Wl, bl):
    src = edge_index[0]
    dst = edge_index[1]

    adj_flat, degp = _get_adj_build()(src, dst)
    adj = adj_flat.reshape(N, N)

    h1 = _gcn_layer_pallas(adj, feats, degp, W1, b1.reshape(1, D))
    cn = _l2norm_pallas(codebook)
    dist, ind2 = _vq_pallas(h1, cn)
    ind = ind2.reshape(N)
    quant = _get_quant_gather()(cn, ind)

    q_edge, sse_commit, sse_node = _dec_pallas(
        quant, h1, Wd1, bd1.reshape(1, D), Wd2, bd2.reshape(1, D))

    stats = _adjq_pallas(q_edge, adj)
    mn, mx, s1, s2, c, a2, a1 = (stats[0], stats[1], stats[2], stats[3],
                                 stats[4], stats[5], stats[6])

    h2, out = _gcn2_pallas(adj, q_edge, degp, W2, b2.reshape(1, D),
                           Wl, bl.reshape(1, DOUT))

    nn = jnp.float32(N) * jnp.float32(N)
    den = mx - mn
    s2n = (s2 - 2.0 * mn * s1 + nn * mn * mn) / (den * den)
    cxn = (c - mn * a1) / den
    edge_rec = jnp.sqrt((a2 - 2.0 * cxn + s2n) / nn)
    feature_rec = sse_node[0, 0] / jnp.float32(N * D)
    commit = 0.25 * sse_commit[0, 0] / jnp.float32(N * D)
    loss = feature_rec + edge_rec + commit

    return (h1, quant, h2, out, loss, dist, cn)


# speedup vs baseline: 2.3602x; 1.2320x over previous
import functools

import jax
import jax.numpy as jnp
from jax import lax
from jax.experimental import pallas as pl
from jax.experimental.pallas import tpu as pltpu
from jax.experimental.pallas import tpu_sc as plsc

N = 4096
E = 65536
D = 128
KC = 8192
DOUT = 64

NW = 32
EP = E // NW
EPC = E // 16
CH_ROWS = 256
CH = CH_ROWS * N
NCH = N // CH_ROWS
STRIPE = CH // 16
ZB = 4096
BI = 256
KJ = 1024


def _adj_build_body(src_hbm, dst_hbm, adj_hbm, deg_hbm,
                    src_v, dst_v, key_g, pos_b, idx_b, val_b, ones_b, zbuf,
                    din_s, dout_s, chunk_s):
    cid = lax.axis_index("c")
    sid = lax.axis_index("s")
    eoff = sid * EPC

    pltpu.sync_copy(src_hbm.at[pl.ds(eoff, EPC)], src_v)
    pltpu.sync_copy(dst_hbm.at[pl.ds(eoff, EPC)], dst_v)

    def _zero(i, _):
        zbuf[pl.ds(i * 16, 16)] = jnp.zeros((16,), jnp.float32)
        return 0
    lax.fori_loop(0, ZB // 16, _zero, 0)

    def _prep(i, _):
        j = i // 8
        c = i % 8
        off = j * 128 + c * 16
        sv = src_v[pl.ds(off, 16)]
        dv = dst_v[pl.ds(off, 16)]
        key_g[j, pl.ds(c * 16, 16)] = dv * N + sv
        pos_b[j, pl.ds(c * 16, 16)] = (
            lax.iota(jnp.int32, 16) + (CH + off))
        ones_b[j, pl.ds(c * 16, 16)] = jnp.ones((16,), jnp.float32)
        return 0
    lax.fori_loop(0, (EPC // 128) * 8, _prep, 0)

    @pl.when(sid == 0)
    def _():
        pltpu.sync_copy(zbuf.at[pl.ds(0, N)], din_s)
        pltpu.sync_copy(zbuf.at[pl.ds(0, N)], dout_s)
    plsc.subcore_barrier()

    def _deg_row(jj, _):
        j = jj + cid * 16
        def _mk_d(c, _2):
            off = j * 128 + c * 16
            idx_b[j, pl.ds(c * 16, 16)] = dst_v[pl.ds(off, 16)]
            return 0
        lax.fori_loop(0, 8, _mk_d, 0)
        pltpu.sync_copy(ones_b.at[j], din_s.at[idx_b.at[j]], add=True)
        def _mk_s(c, _2):
            off = j * 128 + c * 16
            idx_b[j, pl.ds(c * 16, 16)] = src_v[pl.ds(off, 16)]
            return 0
        lax.fori_loop(0, 8, _mk_s, 0)
        pltpu.sync_copy(ones_b.at[j], dout_s.at[idx_b.at[j]], add=True)
        return 0
    lax.fori_loop(0, 16, _deg_row, 0)

    plsc.subcore_barrier()
    @pl.when(sid == 0)
    def _():
        pltpu.sync_copy(din_s, deg_hbm.at[cid, 0])
        pltpu.sync_copy(dout_s, deg_hbm.at[cid, 1])

    for ch in range(NCH // 2):
        chunk_id = ch * 2 + cid
        base = chunk_id * CH

        def _zstripe(k, _):
            pltpu.sync_copy(
                zbuf, chunk_s.at[pl.ds(sid * STRIPE + k * ZB, ZB)])
            return 0
        lax.fori_loop(0, STRIPE // ZB, _zstripe, 0)
        plsc.subcore_barrier()

        def _scat_row(j, _):
            def _mk(c, _2):
                off = j * 128 + c * 16
                k16 = key_g[j, pl.ds(c * 16, 16)] - base
                m = (k16 >= 0) & (k16 < CH)
                idx_b[j, pl.ds(c * 16, 16)] = jnp.where(
                    m, k16, pos_b[j, pl.ds(c * 16, 16)])
                val_b[j, pl.ds(c * 16, 16)] = jnp.where(m, 1.0, 0.0)
                return 0
            lax.fori_loop(0, 8, _mk, 0)
            pltpu.sync_copy(val_b.at[j], chunk_s.at[idx_b.at[j]], add=True)
            return 0
        lax.fori_loop(0, EPC // 128, _scat_row, 0)

        plsc.subcore_barrier()
        pltpu.sync_copy(chunk_s.at[pl.ds(sid * STRIPE, STRIPE)],
                        adj_hbm.at[pl.ds(base + sid * STRIPE, STRIPE)])
        plsc.subcore_barrier()


@functools.cache
def _get_adj_build():
    return functools.partial(
        pl.kernel,
        out_type=[jax.ShapeDtypeStruct((N * N,), jnp.float32),
                  jax.ShapeDtypeStruct((2, 2, N), jnp.float32)],
        mesh=plsc.VectorSubcoreMesh(core_axis_name="c", subcore_axis_name="s"),
        scratch_types=[
        pltpu.VMEM((EPC,), jnp.int32),
        pltpu.VMEM((EPC,), jnp.int32),
        pltpu.VMEM((EPC // 128, 128), jnp.int32),
        pltpu.VMEM((EPC // 128, 128), jnp.int32),
        pltpu.VMEM((EPC // 128, 128), jnp.int32),
        pltpu.VMEM((EPC // 128, 128), jnp.float32),
        pltpu.VMEM((EPC // 128, 128), jnp.float32),
        pltpu.VMEM((ZB,), jnp.float32),
        pltpu.VMEM_SHARED((N,), jnp.float32),
        pltpu.VMEM_SHARED((N,), jnp.float32),
        pltpu.VMEM_SHARED((CH + EPC,), jnp.float32),
        ],
    )(_adj_build_body)


BPW = N // NW


def _quant_gather_body(cn_hbm, ind_hbm, out_hbm, idx_v, rows_v, sem):
    cid = lax.axis_index("c")
    sid = lax.axis_index("s")
    wid = sid * 2 + cid
    base = wid * BPW
    pltpu.sync_copy(ind_hbm.at[pl.ds(base, BPW)], idx_v)
    pltpu.async_copy(cn_hbm.at[idx_v], rows_v, sem).wait()
    pltpu.sync_copy(rows_v, out_hbm.at[pl.ds(base, BPW)])


@functools.cache
def _get_quant_gather():
    return functools.partial(
        pl.kernel,
        out_type=jax.ShapeDtypeStruct((N, D), jnp.float32),
        mesh=plsc.VectorSubcoreMesh(core_axis_name="c", subcore_axis_name="s"),
        scratch_types=[
            pltpu.VMEM((BPW,), jnp.int32),
            pltpu.VMEM((BPW, D), jnp.float32),
            pltpu.SemaphoreType.DMA,
        ],
    )(_quant_gather_body)


def _dup_count_body(adjf_hbm, src_hbm, dst_hbm, out_hbm,
                    src_v, dst_v, idx_k, aval_v, acc_v, sem):
    cid = lax.axis_index("c")
    sid = lax.axis_index("s")
    wid = sid * 2 + cid
    eoff = wid * EP
    pltpu.sync_copy(src_hbm.at[pl.ds(eoff, EP)], src_v)
    pltpu.sync_copy(dst_hbm.at[pl.ds(eoff, EP)], dst_v)
    acc_v[...] = jnp.zeros((16,), jnp.float32)

    def _it(k, _):
        def _f(c, _2):
            off = k * 128 + c * 16
            idx_k[pl.ds(c * 16, 16)] = (
                dst_v[pl.ds(off, 16)] * N + src_v[pl.ds(off, 16)])
            return 0
        lax.fori_loop(0, 8, _f, 0)
        pltpu.async_copy(adjf_hbm.at[idx_k], aval_v, sem).wait()

        def _g(c, _2):
            acc_v[...] = acc_v[...] + aval_v[pl.ds(c * 16, 16)]
            return 0
        lax.fori_loop(0, 8, _g, 0)
        return 0
    lax.fori_loop(0, EP // 128, _it, 0)
    pltpu.sync_copy(acc_v, out_hbm.at[wid])


@functools.cache
def _get_dup_count():
    return functools.partial(
        pl.kernel,
        out_type=jax.ShapeDtypeStruct((NW, 16), jnp.float32),
        mesh=plsc.VectorSubcoreMesh(core_axis_name="c", subcore_axis_name="s"),
        scratch_types=[
            pltpu.VMEM((EP,), jnp.int32),
            pltpu.VMEM((EP,), jnp.int32),
            pltpu.VMEM((128,), jnp.int32),
            pltpu.VMEM((128,), jnp.float32),
            pltpu.VMEM((16,), jnp.float32),
            pltpu.SemaphoreType.DMA,
        ],
    )(_dup_count_body)


def _exact_agg(a, xs):
    f32 = jnp.float32
    bf = jnp.bfloat16
    ah = a.astype(bf)
    al = (a - ah.astype(f32)).astype(bf)
    x1 = xs.astype(bf)
    r1 = xs - x1.astype(f32)
    x2 = r1.astype(bf)
    x3 = (r1 - x2.astype(f32)).astype(bf)
    dot = lambda p, q: jax.lax.dot_general(
        p, q, (((1,), (0,)), ((), ())), preferred_element_type=f32)
    return (dot(ah, x1) + dot(ah, x2)) + (dot(ah, x3) + dot(al, x1))


def _gcn_body(adj_ref, x_ref, degp_ref, w_ref, b_ref, o_ref):
    i = pl.program_id(0)
    deg_out = degp_ref[0, 1, :] + degp_ref[1, 1, :]
    nsrc = lax.rsqrt(jnp.clip(deg_out, 1.0, None))
    deg_in = (degp_ref[0, 0, pl.ds(i * BI, BI)] +
              degp_ref[1, 0, pl.ds(i * BI, BI)])
    ndst = lax.rsqrt(jnp.clip(deg_in, 1.0, None))
    xs = x_ref[...] * nsrc[:, None]
    agg = _exact_agg(adj_ref[...], xs)
    agg = agg * ndst[:, None]
    h = jnp.dot(agg, w_ref[...], preferred_element_type=jnp.float32) + b_ref[...]
    o_ref[...] = jnp.maximum(h, 0.0)


def _gcn_layer_pallas(adj, x, degp, w, b):
    return pl.pallas_call(
        _gcn_body,
        grid=(N // BI,),
        in_specs=[
            pl.BlockSpec((BI, N), lambda i: (i, 0)),
            pl.BlockSpec((N, D), lambda i: (0, 0)),
            pl.BlockSpec((2, 2, N), lambda i: (0, 0, 0)),
            pl.BlockSpec((D, D), lambda i: (0, 0)),
            pl.BlockSpec((1, D), lambda i: (0, 0)),
        ],
        out_specs=pl.BlockSpec((BI, D), lambda i: (i, 0)),
        out_shape=jax.ShapeDtypeStruct((N, D), jnp.float32),
    )(adj, x, degp, w, b)


def _l2_body(x_ref, o_ref):
    x = x_ref[...]
    n = jnp.sqrt(jnp.sum(x * x, axis=1, keepdims=True))
    o_ref[...] = x / jnp.clip(n, 1e-12, None)


def _l2norm_pallas(x):
    m = x.shape[0]
    return pl.pallas_call(
        _l2_body,
        grid=(m // 1024,),
        in_specs=[pl.BlockSpec((1024, D), lambda i: (i, 0))],
        out_specs=pl.BlockSpec((1024, D), lambda i: (i, 0)),
        out_shape=jax.ShapeDtypeStruct((m, D), jnp.float32),
    )(x)


def _vq_body(h1_ref, cn_ref, dist_ref, ind_ref, rmax_ref, ridx_ref, hn_ref):
    j = pl.program_id(1)

    @pl.when(j == 0)
    def _():
        h = h1_ref[...]
        n = jnp.sqrt(jnp.sum(h * h, axis=1, keepdims=True))
        hn_ref[...] = h / jnp.clip(n, 1e-12, None)

    d = lax.dot_general(hn_ref[...], cn_ref[...], (((1,), (1,)), ((), ())),
                        preferred_element_type=jnp.float32)
    dist_ref[...] = d
    tmax = jnp.max(d, axis=1, keepdims=True)
    col = lax.broadcasted_iota(jnp.int32, (BI, KJ), 1)
    tidx = jnp.min(jnp.where(d >= tmax, col, jnp.int32(2 ** 30)),
                   axis=1, keepdims=True) + j * KJ

    @pl.when(j == 0)
    def _():
        rmax_ref[...] = tmax
        ridx_ref[...] = tidx

    @pl.when(j > 0)
    def _():
        better = tmax > rmax_ref[...]
        ridx_ref[...] = jnp.where(better, tidx, ridx_ref[...])
        rmax_ref[...] = jnp.maximum(rmax_ref[...], tmax)

    @pl.when(j == KC // KJ - 1)
    def _():
        ind_ref[...] = ridx_ref[...]


def _vq_pallas(h1, cn):
    return pl.pallas_call(
        _vq_body,
        grid=(N // BI, KC // KJ),
        in_specs=[
            pl.BlockSpec((BI, D), lambda i, j: (i, 0)),
            pl.BlockSpec((KJ, D), lambda i, j: (j, 0)),
        ],
        out_specs=[
            pl.BlockSpec((BI, KJ), lambda i, j: (i, j)),
            pl.BlockSpec((BI, 1), lambda i, j: (i, 0)),
        ],
        out_shape=[
            jax.ShapeDtypeStruct((N, KC), jnp.float32),
            jax.ShapeDtypeStruct((N, 1), jnp.int32),
        ],
        scratch_shapes=[
            pltpu.VMEM((BI, 1), jnp.float32),
            pltpu.VMEM((BI, 1), jnp.int32),
            pltpu.VMEM((BI, D), jnp.float32),
        ],
    )(h1, cn)


def _dec_body(q_ref, h_ref, wd1_ref, bd1_ref, wd2_ref, bd2_ref,
              qe_ref, c_ref, f_ref):
    q = q_ref[...]
    h = h_ref[...]
    qe = lax.dot_general(q, wd1_ref[...], (((1,), (1,)), ((), ())),
                         preferred_element_type=jnp.float32) + bd1_ref[...]
    qn = lax.dot_general(q, wd2_ref[...], (((1,), (1,)), ((), ())),
                         preferred_element_type=jnp.float32) + bd2_ref[...]
    qe_ref[...] = qe
    c_ref[0, 0] = jnp.sum((q - h) ** 2)
    f_ref[0, 0] = jnp.sum((h - qn) ** 2)


def _dec_pallas(quant, h1, wd1, bd1, wd2, bd2):
    return pl.pallas_call(
        _dec_body,
        out_specs=[
            pl.BlockSpec(memory_space=pltpu.VMEM),
            pl.BlockSpec(memory_space=pltpu.SMEM),
            pl.BlockSpec(memory_space=pltpu.SMEM),
        ],
        out_shape=[
            jax.ShapeDtypeStruct((N, D), jnp.float32),
            jax.ShapeDtypeStruct((1, 1), jnp.float32),
            jax.ShapeDtypeStruct((1, 1), jnp.float32),
        ],
    )(quant, h1, wd1, bd1, wd2, bd2)


BQ = 512


def _minmax_body(qei_ref, qej_ref, out_ref):
    i = pl.program_id(0)
    j = pl.program_id(1)
    t = lax.dot_general(qei_ref[...], qej_ref[...], (((1,), (1,)), ((), ())),
                        preferred_element_type=jnp.float32)
    tmn = jnp.min(t)
    tmx = jnp.max(t)
    first = jnp.logical_and(i == 0, j == 0)

    @pl.when(first)
    def _():
        out_ref[0] = tmn
        out_ref[1] = tmx

    @pl.when(jnp.logical_not(first))
    def _():
        out_ref[0] = jnp.minimum(out_ref[0], tmn)
        out_ref[1] = jnp.maximum(out_ref[1], tmx)


def _minmax_pallas(q_edge):
    return pl.pallas_call(
        _minmax_body,
        grid=(N // BQ, N // BQ),
        in_specs=[
            pl.BlockSpec((BQ, D), lambda i, j: (i, 0)),
            pl.BlockSpec((BQ, D), lambda i, j: (j, 0)),
        ],
        out_specs=pl.BlockSpec(memory_space=pltpu.SMEM),
        out_shape=jax.ShapeDtypeStruct((2,), jnp.float32),
    )(q_edge, q_edge)


def _gram_body(q_ref, o_ref):
    q = q_ref[...]
    g = lax.dot_general(q, q, (((0,), (0,)), ((), ())),
                        preferred_element_type=jnp.float32,
                        precision=lax.Precision.HIGHEST)
    cs = jnp.sum(q, axis=0)
    o_ref[0] = jnp.sum(cs * cs)
    o_ref[1] = jnp.sum(g * g)


def _gram_pallas(q_edge):
    return pl.pallas_call(
        _gram_body,
        out_specs=pl.BlockSpec(memory_space=pltpu.SMEM),
        out_shape=jax.ShapeDtypeStruct((2,), jnp.float32),
    )(q_edge)


def _gcn2_body(adj_ref, x_ref, degp_ref, w_ref, b_ref, wl_ref, bl_ref,
               h2_ref, o_ref, c_ref):
    i = pl.program_id(0)
    u = jnp.dot(adj_ref[...], x_ref[...], preferred_element_type=jnp.float32)
    cpart = jnp.sum(x_ref[pl.ds(i * BI, BI), :] * u)

    @pl.when(i == 0)
    def _():
        c_ref[0] = cpart

    @pl.when(i > 0)
    def _():
        c_ref[0] = c_ref[0] + cpart

    deg_out = degp_ref[0, 1, :] + degp_ref[1, 1, :]
    nsrc = lax.rsqrt(jnp.clip(deg_out, 1.0, None))
    deg_in = (degp_ref[0, 0, pl.ds(i * BI, BI)] +
              degp_ref[1, 0, pl.ds(i * BI, BI)])
    ndst = lax.rsqrt(jnp.clip(deg_in, 1.0, None))
    xs = x_ref[...] * nsrc[:, None]
    agg = _exact_agg(adj_ref[...], xs)
    agg = agg * ndst[:, None]
    h = jnp.dot(agg, w_ref[...], preferred_element_type=jnp.float32) + b_ref[...]
    h2 = jnp.maximum(h, 0.0)
    h2_ref[...] = h2
    o_ref[...] = lax.dot_general(h2, wl_ref[...], (((1,), (1,)), ((), ())),
                                 preferred_element_type=jnp.float32) + bl_ref[...]


def _gcn2_pallas(adj, q_edge, degp, w2, b2, wl, bl):
    return pl.pallas_call(
        _gcn2_body,
        grid=(N // BI,),
        in_specs=[
            pl.BlockSpec((BI, N), lambda i: (i, 0)),
            pl.BlockSpec((N, D), lambda i: (0, 0)),
            pl.BlockSpec((2, 2, N), lambda i: (0, 0, 0)),
            pl.BlockSpec((D, D), lambda i: (0, 0)),
            pl.BlockSpec((1, D), lambda i: (0, 0)),
            pl.BlockSpec((DOUT, D), lambda i: (0, 0)),
            pl.BlockSpec((1, DOUT), lambda i: (0, 0)),
        ],
        out_specs=[
            pl.BlockSpec((BI, D), lambda i: (i, 0)),
            pl.BlockSpec((BI, DOUT), lambda i: (i, 0)),
            pl.BlockSpec(memory_space=pltpu.SMEM),
        ],
        out_shape=[
            jax.ShapeDtypeStruct((N, D), jnp.float32),
            jax.ShapeDtypeStruct((N, DOUT), jnp.float32),
            jax.ShapeDtypeStruct((1,), jnp.float32),
        ],
    )(adj, q_edge, degp, w2, b2, wl, bl)


def kernel(feats, edge_index, W1, b1, codebook, Wd1, bd1, Wd2, bd2,
           W2, b2, Wl, bl):
    src = edge_index[0]
    dst = edge_index[1]

    adj_flat, degp = _get_adj_build()(src, dst)
    adj = adj_flat.reshape(N, N)
    dup = _get_dup_count()(adj_flat, src, dst)

    h1 = _gcn_layer_pallas(adj, feats, degp, W1, b1.reshape(1, D))
    cn = _l2norm_pallas(codebook)
    dist, ind2 = _vq_pallas(h1, cn)
    ind = ind2.reshape(N)
    quant = _get_quant_gather()(cn, ind)

    q_edge, sse_commit, sse_node = _dec_pallas(
        quant, h1, Wd1, bd1.reshape(1, D), Wd2, bd2.reshape(1, D))

    mm = _minmax_pallas(q_edge)
    gr = _gram_pallas(q_edge)
    mn, mx, s1, s2 = mm[0], mm[1], gr[0], gr[1]
    a2 = jnp.sum(dup)
    a1 = jnp.float32(E)

    h2, out, cvec = _gcn2_pallas(adj, q_edge, degp, W2, b2.reshape(1, D),
                                 Wl, bl.reshape(1, DOUT))
    c = cvec[0]

    nn = jnp.float32(N) * jnp.float32(N)
    den = mx - mn
    s2n = (s2 - 2.0 * mn * s1 + nn * mn * mn) / (den * den)
    cxn = (c - mn * a1) / den
    edge_rec = jnp.sqrt((a2 - 2.0 * cxn + s2n) / nn)
    feature_rec = sse_node[0, 0] / jnp.float32(N * D)
    commit = 0.25 * sse_commit[0, 0] / jnp.float32(N * D)
    loss = feature_rec + edge_rec + commit

    return (h1, quant, h2, out, loss, dist, cn)


# speedup vs baseline: 2.3995x; 1.0167x over previous
import functools

import jax
import jax.numpy as jnp
from jax import lax
from jax.experimental import pallas as pl
from jax.experimental.pallas import tpu as pltpu
from jax.experimental.pallas import tpu_sc as plsc

N = 4096
E = 65536
D = 128
KC = 8192
DOUT = 64

NW = 32
EP = E // NW
EPC = E // 16
CH_ROWS = 256
CH = CH_ROWS * N
NCH = N // CH_ROWS
STRIPE = CH // 16
ZB = 4096
BI = 256
KJ = 1024


def _adj_build_body(src_hbm, dst_hbm, adj_hbm, deg_hbm,
                    src_v, dst_v, key_g, pos_b, idx_b, val_b, ones_b, zbuf,
                    didx, sidx, din_s, dout_s, chunk_s):
    cid = lax.axis_index("c")
    sid = lax.axis_index("s")
    eoff = sid * EPC

    pltpu.sync_copy(src_hbm.at[pl.ds(eoff, EPC)], src_v)
    pltpu.sync_copy(dst_hbm.at[pl.ds(eoff, EPC)], dst_v)

    def _zero(i, _):
        zbuf[pl.ds(i * 16, 16)] = jnp.zeros((16,), jnp.float32)
        return 0
    lax.fori_loop(0, ZB // 16, _zero, 0)

    def _prep(i, _):
        j = i // 8
        c = i % 8
        off = j * 128 + c * 16
        sv = src_v[pl.ds(off, 16)]
        dv = dst_v[pl.ds(off, 16)]
        key_g[j, pl.ds(c * 16, 16)] = dv * N + sv
        pos_b[j, pl.ds(c * 16, 16)] = (
            lax.iota(jnp.int32, 16) + (CH + off))
        ones_b[j, pl.ds(c * 16, 16)] = jnp.ones((16,), jnp.float32)
        return 0
    lax.fori_loop(0, (EPC // 128) * 8, _prep, 0)

    @pl.when(sid == 0)
    def _():
        pltpu.sync_copy(zbuf.at[pl.ds(0, N)], din_s)
        pltpu.sync_copy(zbuf.at[pl.ds(0, N)], dout_s)
    plsc.subcore_barrier()

    def _deg_row(jj, _):
        def _mk(c, _2):
            off = (jj + cid * 16) * 128 + c * 16
            didx[jj, pl.ds(c * 16, 16)] = dst_v[pl.ds(off, 16)]
            sidx[jj, pl.ds(c * 16, 16)] = src_v[pl.ds(off, 16)]
            return 0
        lax.fori_loop(0, 8, _mk, 0)
        pltpu.sync_copy(ones_b.at[jj], din_s.at[didx.at[jj]], add=True)
        pltpu.sync_copy(ones_b.at[jj], dout_s.at[sidx.at[jj]], add=True)
        return 0
    lax.fori_loop(0, 16, _deg_row, 0)

    plsc.subcore_barrier()
    @pl.when(sid == 0)
    def _():
        pltpu.sync_copy(din_s, deg_hbm.at[cid, 0])
        pltpu.sync_copy(dout_s, deg_hbm.at[cid, 1])

    for ch in range(NCH // 2):
        chunk_id = ch * 2 + cid
        base = chunk_id * CH

        def _zstripe(k, _):
            pltpu.sync_copy(
                zbuf, chunk_s.at[pl.ds(sid * STRIPE + k * ZB, ZB)])
            return 0
        lax.fori_loop(0, STRIPE // ZB, _zstripe, 0)
        plsc.subcore_barrier()

        def _scat_row(j, _):
            def _mk(c, _2):
                k16 = key_g[j, pl.ds(c * 16, 16)] - base
                m = (k16 >= 0) & (k16 < CH)
                idx_b[j, pl.ds(c * 16, 16)] = jnp.where(
                    m, k16, pos_b[j, pl.ds(c * 16, 16)])
                val_b[j, pl.ds(c * 16, 16)] = jnp.where(m, 1.0, 0.0)
                return 0
            lax.fori_loop(0, 8, _mk, 0)
            pltpu.sync_copy(val_b.at[j], chunk_s.at[idx_b.at[j]], add=True)
            return 0
        lax.fori_loop(0, EPC // 128, _scat_row, 0)

        plsc.subcore_barrier()
        pltpu.sync_copy(chunk_s.at[pl.ds(sid * STRIPE, STRIPE)],
                        adj_hbm.at[pl.ds(base + sid * STRIPE, STRIPE)])
        plsc.subcore_barrier()


@functools.cache
def _get_adj_build():
    return functools.partial(
        pl.kernel,
        out_type=[jax.ShapeDtypeStruct((N * N,), jnp.float32),
                  jax.ShapeDtypeStruct((2, 2, N), jnp.float32)],
        mesh=plsc.VectorSubcoreMesh(core_axis_name="c", subcore_axis_name="s"),
        scratch_types=[
        pltpu.VMEM((EPC,), jnp.int32),
        pltpu.VMEM((EPC,), jnp.int32),
        pltpu.VMEM((EPC // 128, 128), jnp.int32),
        pltpu.VMEM((EPC // 128, 128), jnp.int32),
        pltpu.VMEM((EPC // 128, 128), jnp.int32),
        pltpu.VMEM((EPC // 128, 128), jnp.float32),
        pltpu.VMEM((EPC // 128, 128), jnp.float32),
        pltpu.VMEM((ZB,), jnp.float32),
        pltpu.VMEM((16, 128), jnp.int32),
        pltpu.VMEM((16, 128), jnp.int32),
        pltpu.VMEM_SHARED((N,), jnp.float32),
        pltpu.VMEM_SHARED((N,), jnp.float32),
        pltpu.VMEM_SHARED((CH + EPC,), jnp.float32),
        ],
    )(_adj_build_body)


BPW = N // NW


def _quant_gather_body(cn_hbm, ind_hbm, out_hbm, idx_v, rows_v, sem):
    cid = lax.axis_index("c")
    sid = lax.axis_index("s")
    wid = sid * 2 + cid
    base = wid * BPW
    pltpu.sync_copy(ind_hbm.at[pl.ds(base, BPW)], idx_v)
    pltpu.async_copy(cn_hbm.at[idx_v], rows_v, sem).wait()
    pltpu.sync_copy(rows_v, out_hbm.at[pl.ds(base, BPW)])


@functools.cache
def _get_quant_gather():
    return functools.partial(
        pl.kernel,
        out_type=jax.ShapeDtypeStruct((N, D), jnp.float32),
        mesh=plsc.VectorSubcoreMesh(core_axis_name="c", subcore_axis_name="s"),
        scratch_types=[
            pltpu.VMEM((BPW,), jnp.int32),
            pltpu.VMEM((BPW, D), jnp.float32),
            pltpu.SemaphoreType.DMA,
        ],
    )(_quant_gather_body)


def _dup_count_body(adjf_hbm, src_hbm, dst_hbm, out_hbm,
                    src_v, dst_v, idx_k, aval_v, acc_v, sem):
    cid = lax.axis_index("c")
    sid = lax.axis_index("s")
    wid = sid * 2 + cid
    eoff = wid * EP
    pltpu.sync_copy(src_hbm.at[pl.ds(eoff, EP)], src_v)
    pltpu.sync_copy(dst_hbm.at[pl.ds(eoff, EP)], dst_v)
    acc_v[...] = jnp.zeros((16,), jnp.float32)

    def _it(k, _):
        def _f(c, _2):
            off = k * 128 + c * 16
            idx_k[pl.ds(c * 16, 16)] = (
                dst_v[pl.ds(off, 16)] * N + src_v[pl.ds(off, 16)])
            return 0
        lax.fori_loop(0, 8, _f, 0)
        pltpu.async_copy(adjf_hbm.at[idx_k], aval_v, sem).wait()

        def _g(c, _2):
            acc_v[...] = acc_v[...] + aval_v[pl.ds(c * 16, 16)]
            return 0
        lax.fori_loop(0, 8, _g, 0)
        return 0
    lax.fori_loop(0, EP // 128, _it, 0)
    pltpu.sync_copy(acc_v, out_hbm.at[wid])


@functools.cache
def _get_dup_count():
    return functools.partial(
        pl.kernel,
        out_type=jax.ShapeDtypeStruct((NW, 16), jnp.float32),
        mesh=plsc.VectorSubcoreMesh(core_axis_name="c", subcore_axis_name="s"),
        scratch_types=[
            pltpu.VMEM((EP,), jnp.int32),
            pltpu.VMEM((EP,), jnp.int32),
            pltpu.VMEM((128,), jnp.int32),
            pltpu.VMEM((128,), jnp.float32),
            pltpu.VMEM((16,), jnp.float32),
            pltpu.SemaphoreType.DMA,
        ],
    )(_dup_count_body)


def _exact_agg(a, xs):
    f32 = jnp.float32
    bf = jnp.bfloat16
    ah = a.astype(bf)
    al = (a - ah.astype(f32)).astype(bf)
    x1 = xs.astype(bf)
    r1 = xs - x1.astype(f32)
    x2 = r1.astype(bf)
    x3 = (r1 - x2.astype(f32)).astype(bf)
    dot = lambda p, q: jax.lax.dot_general(
        p, q, (((1,), (0,)), ((), ())), preferred_element_type=f32)
    return (dot(ah, x1) + dot(ah, x2)) + (dot(ah, x3) + dot(al, x1))


def _gcn_body(adj_ref, x_ref, degp_ref, w_ref, b_ref, o_ref):
    i = pl.program_id(0)
    deg_out = degp_ref[0, 1, :] + degp_ref[1, 1, :]
    nsrc = lax.rsqrt(jnp.clip(deg_out, 1.0, None))
    deg_in = (degp_ref[0, 0, pl.ds(i * BI, BI)] +
              degp_ref[1, 0, pl.ds(i * BI, BI)])
    ndst = lax.rsqrt(jnp.clip(deg_in, 1.0, None))
    xs = x_ref[...] * nsrc[:, None]
    agg = _exact_agg(adj_ref[...], xs)
    agg = agg * ndst[:, None]
    h = jnp.dot(agg, w_ref[...], preferred_element_type=jnp.float32) + b_ref[...]
    o_ref[...] = jnp.maximum(h, 0.0)


def _gcn_layer_pallas(adj, x, degp, w, b):
    return pl.pallas_call(
        _gcn_body,
        grid=(N // BI,),
        in_specs=[
            pl.BlockSpec((BI, N), lambda i: (i, 0)),
            pl.BlockSpec((N, D), lambda i: (0, 0)),
            pl.BlockSpec((2, 2, N), lambda i: (0, 0, 0)),
            pl.BlockSpec((D, D), lambda i: (0, 0)),
            pl.BlockSpec((1, D), lambda i: (0, 0)),
        ],
        out_specs=pl.BlockSpec((BI, D), lambda i: (i, 0)),
        out_shape=jax.ShapeDtypeStruct((N, D), jnp.float32),
    )(adj, x, degp, w, b)


def _l2_body(x_ref, o_ref):
    x = x_ref[...]
    n = jnp.sqrt(jnp.sum(x * x, axis=1, keepdims=True))
    o_ref[...] = x / jnp.clip(n, 1e-12, None)


def _l2norm_pallas(x):
    m = x.shape[0]
    return pl.pallas_call(
        _l2_body,
        grid=(m // 1024,),
        in_specs=[pl.BlockSpec((1024, D), lambda i: (i, 0))],
        out_specs=pl.BlockSpec((1024, D), lambda i: (i, 0)),
        out_shape=jax.ShapeDtypeStruct((m, D), jnp.float32),
    )(x)


def _vq_body(h1_ref, cn_ref, dist_ref, ind_ref, rmax_ref, ridx_ref, hn_ref):
    j = pl.program_id(1)

    @pl.when(j == 0)
    def _():
        h = h1_ref[...]
        n = jnp.sqrt(jnp.sum(h * h, axis=1, keepdims=True))
        hn_ref[...] = h / jnp.clip(n, 1e-12, None)

    d = lax.dot_general(hn_ref[...], cn_ref[...], (((1,), (1,)), ((), ())),
                        preferred_element_type=jnp.float32)
    dist_ref[...] = d
    tmax = jnp.max(d, axis=1, keepdims=True)
    col = lax.broadcasted_iota(jnp.int32, (BI, KJ), 1)
    tidx = jnp.min(jnp.where(d >= tmax, col, jnp.int32(2 ** 30)),
                   axis=1, keepdims=True) + j * KJ

    @pl.when(j == 0)
    def _():
        rmax_ref[...] = tmax
        ridx_ref[...] = tidx

    @pl.when(j > 0)
    def _():
        better = tmax > rmax_ref[...]
        ridx_ref[...] = jnp.where(better, tidx, ridx_ref[...])
        rmax_ref[...] = jnp.maximum(rmax_ref[...], tmax)

    @pl.when(j == KC // KJ - 1)
    def _():
        ind_ref[...] = ridx_ref[...]


def _vq_pallas(h1, cn):
    return pl.pallas_call(
        _vq_body,
        grid=(N // BI, KC // KJ),
        in_specs=[
            pl.BlockSpec((BI, D), lambda i, j: (i, 0)),
            pl.BlockSpec((KJ, D), lambda i, j: (j, 0)),
        ],
        out_specs=[
            pl.BlockSpec((BI, KJ), lambda i, j: (i, j)),
            pl.BlockSpec((BI, 1), lambda i, j: (i, 0)),
        ],
        out_shape=[
            jax.ShapeDtypeStruct((N, KC), jnp.float32),
            jax.ShapeDtypeStruct((N, 1), jnp.int32),
        ],
        scratch_shapes=[
            pltpu.VMEM((BI, 1), jnp.float32),
            pltpu.VMEM((BI, 1), jnp.int32),
            pltpu.VMEM((BI, D), jnp.float32),
        ],
    )(h1, cn)


def _dec_body(q_ref, h_ref, wd1_ref, bd1_ref, wd2_ref, bd2_ref,
              qe_ref, c_ref, f_ref, g_ref):
    q = q_ref[...]
    h = h_ref[...]
    qe = lax.dot_general(q, wd1_ref[...], (((1,), (1,)), ((), ())),
                         preferred_element_type=jnp.float32) + bd1_ref[...]
    qn = lax.dot_general(q, wd2_ref[...], (((1,), (1,)), ((), ())),
                         preferred_element_type=jnp.float32) + bd2_ref[...]
    qe_ref[...] = qe
    c_ref[0, 0] = jnp.sum((q - h) ** 2)
    f_ref[0, 0] = jnp.sum((h - qn) ** 2)
    g = lax.dot_general(qe, qe, (((0,), (0,)), ((), ())),
                        preferred_element_type=jnp.float32,
                        precision=lax.Precision.HIGHEST)
    cs = jnp.sum(qe, axis=0)
    g_ref[0] = jnp.sum(cs * cs)
    g_ref[1] = jnp.sum(g * g)


def _dec_pallas(quant, h1, wd1, bd1, wd2, bd2):
    return pl.pallas_call(
        _dec_body,
        out_specs=[
            pl.BlockSpec(memory_space=pltpu.VMEM),
            pl.BlockSpec(memory_space=pltpu.SMEM),
            pl.BlockSpec(memory_space=pltpu.SMEM),
            pl.BlockSpec(memory_space=pltpu.SMEM),
        ],
        out_shape=[
            jax.ShapeDtypeStruct((N, D), jnp.float32),
            jax.ShapeDtypeStruct((1, 1), jnp.float32),
            jax.ShapeDtypeStruct((1, 1), jnp.float32),
            jax.ShapeDtypeStruct((2,), jnp.float32),
        ],
    )(quant, h1, wd1, bd1, wd2, bd2)


BQ = 512


def _minmax_body(qei_ref, qej_ref, out_ref):
    i = pl.program_id(0)
    j = pl.program_id(1)

    @pl.when(j >= i)
    def _():
        t = lax.dot_general(qei_ref[...], qej_ref[...],
                            (((1,), (1,)), ((), ())),
                            preferred_element_type=jnp.float32)
        tmn = jnp.min(t)
        tmx = jnp.max(t)
        first = jnp.logical_and(i == 0, j == 0)

        @pl.when(first)
        def _():
            out_ref[0] = tmn
            out_ref[1] = tmx

        @pl.when(jnp.logical_not(first))
        def _():
            out_ref[0] = jnp.minimum(out_ref[0], tmn)
            out_ref[1] = jnp.maximum(out_ref[1], tmx)


def _minmax_pallas(q_edge):
    return pl.pallas_call(
        _minmax_body,
        grid=(N // BQ, N // BQ),
        in_specs=[
            pl.BlockSpec((BQ, D), lambda i, j: (i, 0)),
            pl.BlockSpec((BQ, D), lambda i, j: (j, 0)),
        ],
        out_specs=pl.BlockSpec(memory_space=pltpu.SMEM),
        out_shape=jax.ShapeDtypeStruct((2,), jnp.float32),
    )(q_edge, q_edge)


def _gcn2_body(adj_ref, x_ref, degp_ref, w_ref, b_ref, wl_ref, bl_ref,
               h2_ref, o_ref, c_ref):
    i = pl.program_id(0)
    u = jnp.dot(adj_ref[...], x_ref[...], preferred_element_type=jnp.float32)
    cpart = jnp.sum(x_ref[pl.ds(i * BI, BI), :] * u)

    @pl.when(i == 0)
    def _():
        c_ref[0] = cpart

    @pl.when(i > 0)
    def _():
        c_ref[0] = c_ref[0] + cpart

    deg_out = degp_ref[0, 1, :] + degp_ref[1, 1, :]
    nsrc = lax.rsqrt(jnp.clip(deg_out, 1.0, None))
    deg_in = (degp_ref[0, 0, pl.ds(i * BI, BI)] +
              degp_ref[1, 0, pl.ds(i * BI, BI)])
    ndst = lax.rsqrt(jnp.clip(deg_in, 1.0, None))
    xs = x_ref[...] * nsrc[:, None]
    agg = _exact_agg(adj_ref[...], xs)
    agg = agg * ndst[:, None]
    h = jnp.dot(agg, w_ref[...], preferred_element_type=jnp.float32) + b_ref[...]
    h2 = jnp.maximum(h, 0.0)
    h2_ref[...] = h2
    o_ref[...] = lax.dot_general(h2, wl_ref[...], (((1,), (1,)), ((), ())),
                                 preferred_element_type=jnp.float32) + bl_ref[...]


def _gcn2_pallas(adj, q_edge, degp, w2, b2, wl, bl):
    return pl.pallas_call(
        _gcn2_body,
        grid=(N // BI,),
        in_specs=[
            pl.BlockSpec((BI, N), lambda i: (i, 0)),
            pl.BlockSpec((N, D), lambda i: (0, 0)),
            pl.BlockSpec((2, 2, N), lambda i: (0, 0, 0)),
            pl.BlockSpec((D, D), lambda i: (0, 0)),
            pl.BlockSpec((1, D), lambda i: (0, 0)),
            pl.BlockSpec((DOUT, D), lambda i: (0, 0)),
            pl.BlockSpec((1, DOUT), lambda i: (0, 0)),
        ],
        out_specs=[
            pl.BlockSpec((BI, D), lambda i: (i, 0)),
            pl.BlockSpec((BI, DOUT), lambda i: (i, 0)),
            pl.BlockSpec(memory_space=pltpu.SMEM),
        ],
        out_shape=[
            jax.ShapeDtypeStruct((N, D), jnp.float32),
            jax.ShapeDtypeStruct((N, DOUT), jnp.float32),
            jax.ShapeDtypeStruct((1,), jnp.float32),
        ],
    )(adj, q_edge, degp, w2, b2, wl, bl)


def kernel(feats, edge_index, W1, b1, codebook, Wd1, bd1, Wd2, bd2,
           W2, b2, Wl, bl):
    src = edge_index[0]
    dst = edge_index[1]

    adj_flat, degp = _get_adj_build()(src, dst)
    adj = adj_flat.reshape(N, N)
    dup = _get_dup_count()(adj_flat, src, dst)

    h1 = _gcn_layer_pallas(adj, feats, degp, W1, b1.reshape(1, D))
    cn = _l2norm_pallas(codebook)
    dist, ind2 = _vq_pallas(h1, cn)
    ind = ind2.reshape(N)
    quant = _get_quant_gather()(cn, ind)

    q_edge, sse_commit, sse_node, gr = _dec_pallas(
        quant, h1, Wd1, bd1.reshape(1, D), Wd2, bd2.reshape(1, D))

    mm = _minmax_pallas(q_edge)
    mn, mx, s1, s2 = mm[0], mm[1], gr[0], gr[1]
    a2 = jnp.sum(dup)
    a1 = jnp.float32(E)

    h2, out, cvec = _gcn2_pallas(adj, q_edge, degp, W2, b2.reshape(1, D),
                                 Wl, bl.reshape(1, DOUT))
    c = cvec[0]

    nn = jnp.float32(N) * jnp.float32(N)
    den = mx - mn
    s2n = (s2 - 2.0 * mn * s1 + nn * mn * mn) / (den * den)
    cxn = (c - mn * a1) / den
    edge_rec = jnp.sqrt((a2 - 2.0 * cxn + s2n) / nn)
    feature_rec = sse_node[0, 0] / jnp.float32(N * D)
    commit = 0.25 * sse_commit[0, 0] / jnp.float32(N * D)
    loss = feature_rec + edge_rec + commit

    return (h1, quant, h2, out, loss, dist, cn)


# speedup vs baseline: 2.4927x; 1.0388x over previous
import functools

import jax
import jax.numpy as jnp
from jax import lax
from jax.experimental import pallas as pl
from jax.experimental.pallas import tpu as pltpu
from jax.experimental.pallas import tpu_sc as plsc

N = 4096
E = 65536
D = 128
KC = 8192
DOUT = 64

NW = 32
EP = E // NW
EPC = E // 16
CH_ROWS = 256
CH = CH_ROWS * N
NCH = N // CH_ROWS
STRIPE = CH // 16
ZB = 4096
BI = 256
KJ = 1024


def _adj_build_body(src_hbm, dst_hbm, adj_hbm, deg_hbm,
                    src_v, dst_v, key_g, pos_b, idx_b, val_b, ones_b, zbuf,
                    didx, sidx, din_s, dout_s, chunk_s, sem):
    cid = lax.axis_index("c")
    sid = lax.axis_index("s")
    eoff = sid * EPC

    pltpu.sync_copy(src_hbm.at[pl.ds(eoff, EPC)], src_v)
    pltpu.sync_copy(dst_hbm.at[pl.ds(eoff, EPC)], dst_v)

    def _zero(i, _):
        zbuf[pl.ds(i * 16, 16)] = jnp.zeros((16,), jnp.float32)
        return 0
    lax.fori_loop(0, ZB // 16, _zero, 0)

    def _prep(i, _):
        j = i // 8
        c = i % 8
        off = j * 128 + c * 16
        sv = src_v[pl.ds(off, 16)]
        dv = dst_v[pl.ds(off, 16)]
        key_g[j, pl.ds(c * 16, 16)] = dv * N + sv
        pos_b[j, pl.ds(c * 16, 16)] = (
            lax.iota(jnp.int32, 16) + (CH + off))
        ones_b[j, pl.ds(c * 16, 16)] = jnp.ones((16,), jnp.float32)
        return 0
    lax.fori_loop(0, (EPC // 128) * 8, _prep, 0)

    @pl.when(sid == 0)
    def _():
        pltpu.sync_copy(zbuf.at[pl.ds(0, N)], din_s)
        pltpu.sync_copy(zbuf.at[pl.ds(0, N)], dout_s)
    plsc.subcore_barrier()

    def _deg_fill(i, _):
        jj = i // 8
        c = i % 8
        off = (jj + cid * 16) * 128 + c * 16
        didx[jj, pl.ds(c * 16, 16)] = dst_v[pl.ds(off, 16)]
        sidx[jj, pl.ds(c * 16, 16)] = src_v[pl.ds(off, 16)]
        return 0
    lax.fori_loop(0, 128, _deg_fill, 0)
    dh = ([pltpu.async_copy(ones_b.at[jj], din_s.at[didx.at[jj]], sem,
                            add=True) for jj in range(16)] +
          [pltpu.async_copy(ones_b.at[jj], dout_s.at[sidx.at[jj]], sem,
                            add=True) for jj in range(16)])
    for h in dh:
        h.wait()

    plsc.subcore_barrier()
    @pl.when(sid == 0)
    def _():
        pltpu.sync_copy(din_s, deg_hbm.at[cid, 0])
        pltpu.sync_copy(dout_s, deg_hbm.at[cid, 1])

    for ch in range(NCH // 2):
        chunk_id = ch * 2 + cid
        base = chunk_id * CH

        zh = [pltpu.async_copy(
                  zbuf, chunk_s.at[pl.ds(sid * STRIPE + k * ZB, ZB)], sem)
              for k in range(STRIPE // ZB)]
        for h in zh:
            h.wait()
        plsc.subcore_barrier()

        def _mk_all(i, _):
            j = i // 8
            c = i % 8
            k16 = key_g[j, pl.ds(c * 16, 16)] - base
            m = (k16 >= 0) & (k16 < CH)
            idx_b[j, pl.ds(c * 16, 16)] = jnp.where(
                m, k16, pos_b[j, pl.ds(c * 16, 16)])
            val_b[j, pl.ds(c * 16, 16)] = jnp.where(m, 1.0, 0.0)
            return 0
        lax.fori_loop(0, (EPC // 128) * 8, _mk_all, 0)
        hs = [pltpu.async_copy(val_b.at[j], chunk_s.at[idx_b.at[j]], sem,
                               add=True)
              for j in range(EPC // 128)]
        for h in hs:
            h.wait()

        plsc.subcore_barrier()
        pltpu.sync_copy(chunk_s.at[pl.ds(sid * STRIPE, STRIPE)],
                        adj_hbm.at[pl.ds(base + sid * STRIPE, STRIPE)])
        plsc.subcore_barrier()


@functools.cache
def _get_adj_build():
    return functools.partial(
        pl.kernel,
        out_type=[jax.ShapeDtypeStruct((N * N,), jnp.float32),
                  jax.ShapeDtypeStruct((2, 2, N), jnp.float32)],
        mesh=plsc.VectorSubcoreMesh(core_axis_name="c", subcore_axis_name="s"),
        scratch_types=[
        pltpu.VMEM((EPC,), jnp.int32),
        pltpu.VMEM((EPC,), jnp.int32),
        pltpu.VMEM((EPC // 128, 128), jnp.int32),
        pltpu.VMEM((EPC // 128, 128), jnp.int32),
        pltpu.VMEM((EPC // 128, 128), jnp.int32),
        pltpu.VMEM((EPC // 128, 128), jnp.float32),
        pltpu.VMEM((EPC // 128, 128), jnp.float32),
        pltpu.VMEM((ZB,), jnp.float32),
        pltpu.VMEM((16, 128), jnp.int32),
        pltpu.VMEM((16, 128), jnp.int32),
        pltpu.VMEM_SHARED((N,), jnp.float32),
        pltpu.VMEM_SHARED((N,), jnp.float32),
        pltpu.VMEM_SHARED((CH + EPC,), jnp.float32),
        pltpu.SemaphoreType.DMA,
        ],
    )(_adj_build_body)


BPW = N // NW


def _quant_gather_body(cn_hbm, ind_hbm, out_hbm, idx_v, rows_v, sem):
    cid = lax.axis_index("c")
    sid = lax.axis_index("s")
    wid = sid * 2 + cid
    base = wid * BPW
    pltpu.sync_copy(ind_hbm.at[pl.ds(base, BPW)], idx_v)
    pltpu.async_copy(cn_hbm.at[idx_v], rows_v, sem).wait()
    pltpu.sync_copy(rows_v, out_hbm.at[pl.ds(base, BPW)])


@functools.cache
def _get_quant_gather():
    return functools.partial(
        pl.kernel,
        out_type=jax.ShapeDtypeStruct((N, D), jnp.float32),
        mesh=plsc.VectorSubcoreMesh(core_axis_name="c", subcore_axis_name="s"),
        scratch_types=[
            pltpu.VMEM((BPW,), jnp.int32),
            pltpu.VMEM((BPW, D), jnp.float32),
            pltpu.SemaphoreType.DMA,
        ],
    )(_quant_gather_body)


def _dup_count_body(adjf_hbm, src_hbm, dst_hbm, out_hbm,
                    src_v, dst_v, idx_k, aval_v, acc_v, sem):
    cid = lax.axis_index("c")
    sid = lax.axis_index("s")
    wid = sid * 2 + cid
    eoff = wid * EP
    pltpu.sync_copy(src_hbm.at[pl.ds(eoff, EP)], src_v)
    pltpu.sync_copy(dst_hbm.at[pl.ds(eoff, EP)], dst_v)
    acc_v[...] = jnp.zeros((16,), jnp.float32)

    def _it(k, _):
        def _f(c, _2):
            off = k * 128 + c * 16
            idx_k[pl.ds(c * 16, 16)] = (
                dst_v[pl.ds(off, 16)] * N + src_v[pl.ds(off, 16)])
            return 0
        lax.fori_loop(0, 8, _f, 0)
        pltpu.async_copy(adjf_hbm.at[idx_k], aval_v, sem).wait()

        def _g(c, _2):
            acc_v[...] = acc_v[...] + aval_v[pl.ds(c * 16, 16)]
            return 0
        lax.fori_loop(0, 8, _g, 0)
        return 0
    lax.fori_loop(0, EP // 128, _it, 0)
    pltpu.sync_copy(acc_v, out_hbm.at[wid])


@functools.cache
def _get_dup_count():
    return functools.partial(
        pl.kernel,
        out_type=jax.ShapeDtypeStruct((NW, 16), jnp.float32),
        mesh=plsc.VectorSubcoreMesh(core_axis_name="c", subcore_axis_name="s"),
        scratch_types=[
            pltpu.VMEM((EP,), jnp.int32),
            pltpu.VMEM((EP,), jnp.int32),
            pltpu.VMEM((128,), jnp.int32),
            pltpu.VMEM((128,), jnp.float32),
            pltpu.VMEM((16,), jnp.float32),
            pltpu.SemaphoreType.DMA,
        ],
    )(_dup_count_body)


def _exact_agg(a, xs):
    f32 = jnp.float32
    bf = jnp.bfloat16
    ah = a.astype(bf)
    al = (a - ah.astype(f32)).astype(bf)
    x1 = xs.astype(bf)
    r1 = xs - x1.astype(f32)
    x2 = r1.astype(bf)
    x3 = (r1 - x2.astype(f32)).astype(bf)
    dot = lambda p, q: jax.lax.dot_general(
        p, q, (((1,), (0,)), ((), ())), preferred_element_type=f32)
    return (dot(ah, x1) + dot(ah, x2)) + (dot(ah, x3) + dot(al, x1))


def _gcn_body(adj_ref, x_ref, degp_ref, w_ref, b_ref, o_ref):
    i = pl.program_id(0)
    deg_out = degp_ref[0, 1, :] + degp_ref[1, 1, :]
    nsrc = lax.rsqrt(jnp.clip(deg_out, 1.0, None))
    deg_in = (degp_ref[0, 0, pl.ds(i * BI, BI)] +
              degp_ref[1, 0, pl.ds(i * BI, BI)])
    ndst = lax.rsqrt(jnp.clip(deg_in, 1.0, None))
    xs = x_ref[...] * nsrc[:, None]
    agg = _exact_agg(adj_ref[...], xs)
    agg = agg * ndst[:, None]
    h = jnp.dot(agg, w_ref[...], preferred_element_type=jnp.float32) + b_ref[...]
    o_ref[...] = jnp.maximum(h, 0.0)


def _gcn_layer_pallas(adj, x, degp, w, b):
    return pl.pallas_call(
        _gcn_body,
        grid=(N // BI,),
        in_specs=[
            pl.BlockSpec((BI, N), lambda i: (i, 0)),
            pl.BlockSpec((N, D), lambda i: (0, 0)),
            pl.BlockSpec((2, 2, N), lambda i: (0, 0, 0)),
            pl.BlockSpec((D, D), lambda i: (0, 0)),
            pl.BlockSpec((1, D), lambda i: (0, 0)),
        ],
        out_specs=pl.BlockSpec((BI, D), lambda i: (i, 0)),
        out_shape=jax.ShapeDtypeStruct((N, D), jnp.float32),
    )(adj, x, degp, w, b)


def _l2_body(x_ref, o_ref):
    x = x_ref[...]
    n = jnp.sqrt(jnp.sum(x * x, axis=1, keepdims=True))
    o_ref[...] = x / jnp.clip(n, 1e-12, None)


def _l2norm_pallas(x):
    m = x.shape[0]
    return pl.pallas_call(
        _l2_body,
        grid=(m // 1024,),
        in_specs=[pl.BlockSpec((1024, D), lambda i: (i, 0))],
        out_specs=pl.BlockSpec((1024, D), lambda i: (i, 0)),
        out_shape=jax.ShapeDtypeStruct((m, D), jnp.float32),
    )(x)


def _vq_body(h1_ref, cn_ref, dist_ref, ind_ref, rmax_ref, ridx_ref, hn_ref):
    j = pl.program_id(1)

    @pl.when(j == 0)
    def _():
        h = h1_ref[...]
        n = jnp.sqrt(jnp.sum(h * h, axis=1, keepdims=True))
        hn_ref[...] = h / jnp.clip(n, 1e-12, None)

    d = lax.dot_general(hn_ref[...], cn_ref[...], (((1,), (1,)), ((), ())),
                        preferred_element_type=jnp.float32)
    dist_ref[...] = d
    tmax = jnp.max(d, axis=1, keepdims=True)
    col = lax.broadcasted_iota(jnp.int32, (BI, KJ), 1)
    tidx = jnp.min(jnp.where(d >= tmax, col, jnp.int32(2 ** 30)),
                   axis=1, keepdims=True) + j * KJ

    @pl.when(j == 0)
    def _():
        rmax_ref[...] = tmax
        ridx_ref[...] = tidx

    @pl.when(j > 0)
    def _():
        better = tmax > rmax_ref[...]
        ridx_ref[...] = jnp.where(better, tidx, ridx_ref[...])
        rmax_ref[...] = jnp.maximum(rmax_ref[...], tmax)

    @pl.when(j == KC // KJ - 1)
    def _():
        ind_ref[...] = ridx_ref[...]


def _vq_pallas(h1, cn):
    return pl.pallas_call(
        _vq_body,
        grid=(N // BI, KC // KJ),
        in_specs=[
            pl.BlockSpec((BI, D), lambda i, j: (i, 0)),
            pl.BlockSpec((KJ, D), lambda i, j: (j, 0)),
        ],
        out_specs=[
            pl.BlockSpec((BI, KJ), lambda i, j: (i, j)),
            pl.BlockSpec((BI, 1), lambda i, j: (i, 0)),
        ],
        out_shape=[
            jax.ShapeDtypeStruct((N, KC), jnp.float32),
            jax.ShapeDtypeStruct((N, 1), jnp.int32),
        ],
        scratch_shapes=[
            pltpu.VMEM((BI, 1), jnp.float32),
            pltpu.VMEM((BI, 1), jnp.int32),
            pltpu.VMEM((BI, D), jnp.float32),
        ],
    )(h1, cn)


def _dec_body(q_ref, h_ref, wd1_ref, bd1_ref, wd2_ref, bd2_ref,
              qe_ref, c_ref, f_ref, g_ref):
    q = q_ref[...]
    h = h_ref[...]
    qe = lax.dot_general(q, wd1_ref[...], (((1,), (1,)), ((), ())),
                         preferred_element_type=jnp.float32) + bd1_ref[...]
    qn = lax.dot_general(q, wd2_ref[...], (((1,), (1,)), ((), ())),
                         preferred_element_type=jnp.float32) + bd2_ref[...]
    qe_ref[...] = qe
    c_ref[0, 0] = jnp.sum((q - h) ** 2)
    f_ref[0, 0] = jnp.sum((h - qn) ** 2)
    g = lax.dot_general(qe, qe, (((0,), (0,)), ((), ())),
                        preferred_element_type=jnp.float32,
                        precision=lax.Precision.HIGHEST)
    cs = jnp.sum(qe, axis=0)
    g_ref[0] = jnp.sum(cs * cs)
    g_ref[1] = jnp.sum(g * g)


def _dec_pallas(quant, h1, wd1, bd1, wd2, bd2):
    return pl.pallas_call(
        _dec_body,
        out_specs=[
            pl.BlockSpec(memory_space=pltpu.VMEM),
            pl.BlockSpec(memory_space=pltpu.SMEM),
            pl.BlockSpec(memory_space=pltpu.SMEM),
            pl.BlockSpec(memory_space=pltpu.SMEM),
        ],
        out_shape=[
            jax.ShapeDtypeStruct((N, D), jnp.float32),
            jax.ShapeDtypeStruct((1, 1), jnp.float32),
            jax.ShapeDtypeStruct((1, 1), jnp.float32),
            jax.ShapeDtypeStruct((2,), jnp.float32),
        ],
    )(quant, h1, wd1, bd1, wd2, bd2)


BQ = 512


def _minmax_body(qei_ref, qej_ref, out_ref):
    i = pl.program_id(0)
    j = pl.program_id(1)

    @pl.when(j >= i)
    def _():
        t = lax.dot_general(qei_ref[...], qej_ref[...],
                            (((1,), (1,)), ((), ())),
                            preferred_element_type=jnp.float32)
        tmn = jnp.min(t)
        tmx = jnp.max(t)
        first = jnp.logical_and(i == 0, j == 0)

        @pl.when(first)
        def _():
            out_ref[0] = tmn
            out_ref[1] = tmx

        @pl.when(jnp.logical_not(first))
        def _():
            out_ref[0] = jnp.minimum(out_ref[0], tmn)
            out_ref[1] = jnp.maximum(out_ref[1], tmx)


def _minmax_pallas(q_edge):
    return pl.pallas_call(
        _minmax_body,
        grid=(N // BQ, N // BQ),
        in_specs=[
            pl.BlockSpec((BQ, D), lambda i, j: (i, 0)),
            pl.BlockSpec((BQ, D), lambda i, j: (j, 0)),
        ],
        out_specs=pl.BlockSpec(memory_space=pltpu.SMEM),
        out_shape=jax.ShapeDtypeStruct((2,), jnp.float32),
    )(q_edge, q_edge)


def _gcn2_body(adj_ref, x_ref, degp_ref, w_ref, b_ref, wl_ref, bl_ref,
               h2_ref, o_ref, c_ref):
    i = pl.program_id(0)
    u = jnp.dot(adj_ref[...], x_ref[...], preferred_element_type=jnp.float32)
    cpart = jnp.sum(x_ref[pl.ds(i * BI, BI), :] * u)

    @pl.when(i == 0)
    def _():
        c_ref[0] = cpart

    @pl.when(i > 0)
    def _():
        c_ref[0] = c_ref[0] + cpart

    deg_out = degp_ref[0, 1, :] + degp_ref[1, 1, :]
    nsrc = lax.rsqrt(jnp.clip(deg_out, 1.0, None))
    deg_in = (degp_ref[0, 0, pl.ds(i * BI, BI)] +
              degp_ref[1, 0, pl.ds(i * BI, BI)])
    ndst = lax.rsqrt(jnp.clip(deg_in, 1.0, None))
    xs = x_ref[...] * nsrc[:, None]
    agg = _exact_agg(adj_ref[...], xs)
    agg = agg * ndst[:, None]
    h = jnp.dot(agg, w_ref[...], preferred_element_type=jnp.float32) + b_ref[...]
    h2 = jnp.maximum(h, 0.0)
    h2_ref[...] = h2
    o_ref[...] = lax.dot_general(h2, wl_ref[...], (((1,), (1,)), ((), ())),
                                 preferred_element_type=jnp.float32) + bl_ref[...]


def _gcn2_pallas(adj, q_edge, degp, w2, b2, wl, bl):
    return pl.pallas_call(
        _gcn2_body,
        grid=(N // BI,),
        in_specs=[
            pl.BlockSpec((BI, N), lambda i: (i, 0)),
            pl.BlockSpec((N, D), lambda i: (0, 0)),
            pl.BlockSpec((2, 2, N), lambda i: (0, 0, 0)),
            pl.BlockSpec((D, D), lambda i: (0, 0)),
            pl.BlockSpec((1, D), lambda i: (0, 0)),
            pl.BlockSpec((DOUT, D), lambda i: (0, 0)),
            pl.BlockSpec((1, DOUT), lambda i: (0, 0)),
        ],
        out_specs=[
            pl.BlockSpec((BI, D), lambda i: (i, 0)),
            pl.BlockSpec((BI, DOUT), lambda i: (i, 0)),
            pl.BlockSpec(memory_space=pltpu.SMEM),
        ],
        out_shape=[
            jax.ShapeDtypeStruct((N, D), jnp.float32),
            jax.ShapeDtypeStruct((N, DOUT), jnp.float32),
            jax.ShapeDtypeStruct((1,), jnp.float32),
        ],
    )(adj, q_edge, degp, w2, b2, wl, bl)


def kernel(feats, edge_index, W1, b1, codebook, Wd1, bd1, Wd2, bd2,
           W2, b2, Wl, bl):
    src = edge_index[0]
    dst = edge_index[1]

    adj_flat, degp = _get_adj_build()(src, dst)
    adj = adj_flat.reshape(N, N)
    dup = _get_dup_count()(adj_flat, src, dst)

    h1 = _gcn_layer_pallas(adj, feats, degp, W1, b1.reshape(1, D))
    cn = _l2norm_pallas(codebook)
    dist, ind2 = _vq_pallas(h1, cn)
    ind = ind2.reshape(N)
    quant = _get_quant_gather()(cn, ind)

    q_edge, sse_commit, sse_node, gr = _dec_pallas(
        quant, h1, Wd1, bd1.reshape(1, D), Wd2, bd2.reshape(1, D))

    mm = _minmax_pallas(q_edge)
    mn, mx, s1, s2 = mm[0], mm[1], gr[0], gr[1]
    a2 = jnp.sum(dup)
    a1 = jnp.float32(E)

    h2, out, cvec = _gcn2_pallas(adj, q_edge, degp, W2, b2.reshape(1, D),
                                 Wl, bl.reshape(1, DOUT))
    c = cvec[0]

    nn = jnp.float32(N) * jnp.float32(N)
    den = mx - mn
    s2n = (s2 - 2.0 * mn * s1 + nn * mn * mn) / (den * den)
    cxn = (c - mn * a1) / den
    edge_rec = jnp.sqrt((a2 - 2.0 * cxn + s2n) / nn)
    feature_rec = sse_node[0, 0] / jnp.float32(N * D)
    commit = 0.25 * sse_commit[0, 0] / jnp.float32(N * D)
    loss = feature_rec + edge_rec + commit

    return (h1, quant, h2, out, loss, dist, cn)


# speedup vs baseline: 2.6433x; 1.0604x over previous
import functools

import jax
import jax.numpy as jnp
from jax import lax
from jax.experimental import pallas as pl
from jax.experimental.pallas import tpu as pltpu
from jax.experimental.pallas import tpu_sc as plsc

N = 4096
E = 65536
D = 128
KC = 8192
DOUT = 64

NW = 32
EP = E // NW
EPC = E // 16
CH_ROWS = 256
CH = CH_ROWS * N
NCH = N // CH_ROWS
STRIPE = CH // 16
ZB = 4096
BI = 256
KJ = 1024


def _adj_build_body(src_hbm, dst_hbm, adj_hbm, deg_hbm,
                    src_v, dst_v, key_g, pos_b, idx_b, val_b, ones_b, zbuf,
                    didx, sidx, din_s, dout_s, chunk_s, sem):
    cid = lax.axis_index("c")
    sid = lax.axis_index("s")
    eoff = sid * EPC

    pltpu.sync_copy(src_hbm.at[pl.ds(eoff, EPC)], src_v)
    pltpu.sync_copy(dst_hbm.at[pl.ds(eoff, EPC)], dst_v)

    def _zero(i, _):
        zbuf[pl.ds(i * 16, 16)] = jnp.zeros((16,), jnp.float32)
        return 0
    lax.fori_loop(0, ZB // 16, _zero, 0)

    def _prep(i, _):
        j = i // 8
        c = i % 8
        off = j * 128 + c * 16
        sv = src_v[pl.ds(off, 16)]
        dv = dst_v[pl.ds(off, 16)]
        key_g[j, pl.ds(c * 16, 16)] = dv * N + sv
        pos_b[j, pl.ds(c * 16, 16)] = (
            lax.iota(jnp.int32, 16) + (CH + off))
        ones_b[j, pl.ds(c * 16, 16)] = jnp.ones((16,), jnp.float32)
        return 0
    lax.fori_loop(0, (EPC // 128) * 8, _prep, 0)

    @pl.when(sid == 0)
    def _():
        pltpu.sync_copy(zbuf.at[pl.ds(0, N)], din_s)
        pltpu.sync_copy(zbuf.at[pl.ds(0, N)], dout_s)
    plsc.subcore_barrier()

    def _deg_fill(i, _):
        jj = i // 8
        c = i % 8
        off = (jj + cid * 16) * 128 + c * 16
        didx[jj, pl.ds(c * 16, 16)] = dst_v[pl.ds(off, 16)]
        sidx[jj, pl.ds(c * 16, 16)] = src_v[pl.ds(off, 16)]
        return 0
    lax.fori_loop(0, 128, _deg_fill, 0)
    dh = ([pltpu.async_copy(ones_b.at[jj], din_s.at[didx.at[jj]], sem,
                            add=True) for jj in range(16)] +
          [pltpu.async_copy(ones_b.at[jj], dout_s.at[sidx.at[jj]], sem,
                            add=True) for jj in range(16)])
    for h in dh:
        h.wait()

    plsc.subcore_barrier()
    @pl.when(sid == 0)
    def _():
        pltpu.sync_copy(din_s, deg_hbm.at[cid, 0])
        pltpu.sync_copy(dout_s, deg_hbm.at[cid, 1])

    for ch in range(NCH // 2):
        chunk_id = ch * 2 + cid
        base = chunk_id * CH

        zh = [pltpu.async_copy(
                  zbuf, chunk_s.at[pl.ds(sid * STRIPE + k * ZB, ZB)], sem)
              for k in range(STRIPE // ZB)]
        for h in zh:
            h.wait()
        plsc.subcore_barrier()

        def _mk_all(i, _):
            j = i // 8
            c = i % 8
            k16 = key_g[j, pl.ds(c * 16, 16)] - base
            m = (k16 >= 0) & (k16 < CH)
            idx_b[j, pl.ds(c * 16, 16)] = jnp.where(
                m, k16, pos_b[j, pl.ds(c * 16, 16)])
            val_b[j, pl.ds(c * 16, 16)] = jnp.where(m, 1.0, 0.0)
            return 0
        lax.fori_loop(0, (EPC // 128) * 8, _mk_all, 0)
        hs = [pltpu.async_copy(val_b.at[j], chunk_s.at[idx_b.at[j]], sem,
                               add=True)
              for j in range(EPC // 128)]
        for h in hs:
            h.wait()

        plsc.subcore_barrier()
        pltpu.sync_copy(chunk_s.at[pl.ds(sid * STRIPE, STRIPE)],
                        adj_hbm.at[pl.ds(base + sid * STRIPE, STRIPE)])
        plsc.subcore_barrier()


@functools.cache
def _get_adj_build():
    return functools.partial(
        pl.kernel,
        out_type=[jax.ShapeDtypeStruct((N * N,), jnp.float32),
                  jax.ShapeDtypeStruct((2, 2, N), jnp.float32)],
        mesh=plsc.VectorSubcoreMesh(core_axis_name="c", subcore_axis_name="s"),
        scratch_types=[
        pltpu.VMEM((EPC,), jnp.int32),
        pltpu.VMEM((EPC,), jnp.int32),
        pltpu.VMEM((EPC // 128, 128), jnp.int32),
        pltpu.VMEM((EPC // 128, 128), jnp.int32),
        pltpu.VMEM((EPC // 128, 128), jnp.int32),
        pltpu.VMEM((EPC // 128, 128), jnp.float32),
        pltpu.VMEM((EPC // 128, 128), jnp.float32),
        pltpu.VMEM((ZB,), jnp.float32),
        pltpu.VMEM((16, 128), jnp.int32),
        pltpu.VMEM((16, 128), jnp.int32),
        pltpu.VMEM_SHARED((N,), jnp.float32),
        pltpu.VMEM_SHARED((N,), jnp.float32),
        pltpu.VMEM_SHARED((CH + EPC,), jnp.float32),
        pltpu.SemaphoreType.DMA,
        ],
    )(_adj_build_body)


BPW = N // NW


def _quant_gather_body(cn_hbm, ind_hbm, out_hbm, idx_v, rows_v, sem):
    cid = lax.axis_index("c")
    sid = lax.axis_index("s")
    wid = sid * 2 + cid
    base = wid * BPW
    pltpu.sync_copy(ind_hbm.at[pl.ds(base, BPW)], idx_v)
    pltpu.async_copy(cn_hbm.at[idx_v], rows_v, sem).wait()
    pltpu.sync_copy(rows_v, out_hbm.at[pl.ds(base, BPW)])


@functools.cache
def _get_quant_gather():
    return functools.partial(
        pl.kernel,
        out_type=jax.ShapeDtypeStruct((N, D), jnp.float32),
        mesh=plsc.VectorSubcoreMesh(core_axis_name="c", subcore_axis_name="s"),
        scratch_types=[
            pltpu.VMEM((BPW,), jnp.int32),
            pltpu.VMEM((BPW, D), jnp.float32),
            pltpu.SemaphoreType.DMA,
        ],
    )(_quant_gather_body)


def _dup_count_body(adjf_hbm, src_hbm, dst_hbm, out_hbm,
                    src_v, dst_v, idx_k, aval_v, acc_v, sem):
    cid = lax.axis_index("c")
    sid = lax.axis_index("s")
    wid = sid * 2 + cid
    eoff = wid * EP
    pltpu.sync_copy(src_hbm.at[pl.ds(eoff, EP)], src_v)
    pltpu.sync_copy(dst_hbm.at[pl.ds(eoff, EP)], dst_v)
    acc_v[...] = jnp.zeros((16,), jnp.float32)

    def _it(k, _):
        def _f(c, _2):
            off = k * 128 + c * 16
            idx_k[pl.ds(c * 16, 16)] = (
                dst_v[pl.ds(off, 16)] * N + src_v[pl.ds(off, 16)])
            return 0
        lax.fori_loop(0, 8, _f, 0)
        pltpu.async_copy(adjf_hbm.at[idx_k], aval_v, sem).wait()

        def _g(c, _2):
            acc_v[...] = acc_v[...] + aval_v[pl.ds(c * 16, 16)]
            return 0
        lax.fori_loop(0, 8, _g, 0)
        return 0
    lax.fori_loop(0, EP // 128, _it, 0)
    pltpu.sync_copy(acc_v, out_hbm.at[wid])


@functools.cache
def _get_dup_count():
    return functools.partial(
        pl.kernel,
        out_type=jax.ShapeDtypeStruct((NW, 16), jnp.float32),
        mesh=plsc.VectorSubcoreMesh(core_axis_name="c", subcore_axis_name="s"),
        scratch_types=[
            pltpu.VMEM((EP,), jnp.int32),
            pltpu.VMEM((EP,), jnp.int32),
            pltpu.VMEM((128,), jnp.int32),
            pltpu.VMEM((128,), jnp.float32),
            pltpu.VMEM((16,), jnp.float32),
            pltpu.SemaphoreType.DMA,
        ],
    )(_dup_count_body)


def _exact_agg(a, xs):
    f32 = jnp.float32
    bf = jnp.bfloat16
    ah = a.astype(bf)
    al = (a - ah.astype(f32)).astype(bf)
    x1 = xs.astype(bf)
    r1 = xs - x1.astype(f32)
    x2 = r1.astype(bf)
    x3 = (r1 - x2.astype(f32)).astype(bf)
    dot = lambda p, q: jax.lax.dot_general(
        p, q, (((1,), (0,)), ((), ())), preferred_element_type=f32)
    return (dot(ah, x1) + dot(ah, x2)) + (dot(ah, x3) + dot(al, x1))


def _gcn_body(adj_ref, x_ref, degp_ref, w_ref, b_ref, o_ref):
    i = pl.program_id(0)
    deg_out = degp_ref[0, 1, :] + degp_ref[1, 1, :]
    nsrc = lax.rsqrt(jnp.clip(deg_out, 1.0, None))
    deg_in = (degp_ref[0, 0, pl.ds(i * BI, BI)] +
              degp_ref[1, 0, pl.ds(i * BI, BI)])
    ndst = lax.rsqrt(jnp.clip(deg_in, 1.0, None))
    xs = x_ref[...] * nsrc[:, None]
    agg = _exact_agg(adj_ref[...], xs)
    agg = agg * ndst[:, None]
    h = jnp.dot(agg, w_ref[...], preferred_element_type=jnp.float32) + b_ref[...]
    o_ref[...] = jnp.maximum(h, 0.0)


def _gcn_layer_pallas(adj, x, degp, w, b):
    return pl.pallas_call(
        _gcn_body,
        grid=(N // BI,),
        in_specs=[
            pl.BlockSpec((BI, N), lambda i: (i, 0)),
            pl.BlockSpec((N, D), lambda i: (0, 0)),
            pl.BlockSpec((2, 2, N), lambda i: (0, 0, 0)),
            pl.BlockSpec((D, D), lambda i: (0, 0)),
            pl.BlockSpec((1, D), lambda i: (0, 0)),
        ],
        out_specs=pl.BlockSpec((BI, D), lambda i: (i, 0)),
        out_shape=jax.ShapeDtypeStruct((N, D), jnp.float32),
    )(adj, x, degp, w, b)


def _l2_body(x_ref, o_ref):
    x = x_ref[...]
    n = jnp.sqrt(jnp.sum(x * x, axis=1, keepdims=True))
    o_ref[...] = x / jnp.clip(n, 1e-12, None)


def _l2norm_pallas(x):
    m = x.shape[0]
    return pl.pallas_call(
        _l2_body,
        grid=(m // 1024,),
        in_specs=[pl.BlockSpec((1024, D), lambda i: (i, 0))],
        out_specs=pl.BlockSpec((1024, D), lambda i: (i, 0)),
        out_shape=jax.ShapeDtypeStruct((m, D), jnp.float32),
    )(x)


def _vq_body(h1_ref, cn_ref, dist_ref, ind_ref, rmax_ref, ridx_ref, hn_ref):
    j = pl.program_id(1)

    @pl.when(j == 0)
    def _():
        h = h1_ref[...]
        n = jnp.sqrt(jnp.sum(h * h, axis=1, keepdims=True))
        hn_ref[...] = h / jnp.clip(n, 1e-12, None)

    d = lax.dot_general(hn_ref[...], cn_ref[pl.ds(j * KJ, KJ), :],
                        (((1,), (1,)), ((), ())),
                        preferred_element_type=jnp.float32)
    dist_ref[...] = d
    tmax = jnp.max(d, axis=1, keepdims=True)
    col = lax.broadcasted_iota(jnp.int32, (BI, KJ), 1)
    tidx = jnp.min(jnp.where(d >= tmax, col, jnp.int32(2 ** 30)),
                   axis=1, keepdims=True) + j * KJ

    @pl.when(j == 0)
    def _():
        rmax_ref[...] = tmax
        ridx_ref[...] = tidx

    @pl.when(j > 0)
    def _():
        better = tmax > rmax_ref[...]
        ridx_ref[...] = jnp.where(better, tidx, ridx_ref[...])
        rmax_ref[...] = jnp.maximum(rmax_ref[...], tmax)

    @pl.when(j == KC // KJ - 1)
    def _():
        ind_ref[...] = ridx_ref[...]


def _vq_pallas(h1, cn):
    return pl.pallas_call(
        _vq_body,
        grid=(N // BI, KC // KJ),
        in_specs=[
            pl.BlockSpec((BI, D), lambda i, j: (i, 0)),
            pl.BlockSpec((KC, D), lambda i, j: (0, 0)),
        ],
        out_specs=[
            pl.BlockSpec((BI, KJ), lambda i, j: (i, j)),
            pl.BlockSpec((BI, 1), lambda i, j: (i, 0)),
        ],
        out_shape=[
            jax.ShapeDtypeStruct((N, KC), jnp.float32),
            jax.ShapeDtypeStruct((N, 1), jnp.int32),
        ],
        scratch_shapes=[
            pltpu.VMEM((BI, 1), jnp.float32),
            pltpu.VMEM((BI, 1), jnp.int32),
            pltpu.VMEM((BI, D), jnp.float32),
        ],
    )(h1, cn)


def _dec_body(q_ref, h_ref, wd1_ref, bd1_ref, wd2_ref, bd2_ref,
              qe_ref, c_ref, f_ref, g_ref):
    q = q_ref[...]
    h = h_ref[...]
    qe = lax.dot_general(q, wd1_ref[...], (((1,), (1,)), ((), ())),
                         preferred_element_type=jnp.float32) + bd1_ref[...]
    qn = lax.dot_general(q, wd2_ref[...], (((1,), (1,)), ((), ())),
                         preferred_element_type=jnp.float32) + bd2_ref[...]
    qe_ref[...] = qe
    c_ref[0, 0] = jnp.sum((q - h) ** 2)
    f_ref[0, 0] = jnp.sum((h - qn) ** 2)
    g = lax.dot_general(qe, qe, (((0,), (0,)), ((), ())),
                        preferred_element_type=jnp.float32,
                        precision=lax.Precision.HIGHEST)
    cs = jnp.sum(qe, axis=0)
    g_ref[0] = jnp.sum(cs * cs)
    g_ref[1] = jnp.sum(g * g)


def _dec_pallas(quant, h1, wd1, bd1, wd2, bd2):
    return pl.pallas_call(
        _dec_body,
        out_specs=[
            pl.BlockSpec(memory_space=pltpu.VMEM),
            pl.BlockSpec(memory_space=pltpu.SMEM),
            pl.BlockSpec(memory_space=pltpu.SMEM),
            pl.BlockSpec(memory_space=pltpu.SMEM),
        ],
        out_shape=[
            jax.ShapeDtypeStruct((N, D), jnp.float32),
            jax.ShapeDtypeStruct((1, 1), jnp.float32),
            jax.ShapeDtypeStruct((1, 1), jnp.float32),
            jax.ShapeDtypeStruct((2,), jnp.float32),
        ],
    )(quant, h1, wd1, bd1, wd2, bd2)


BQ = 512


def _minmax_body(qei_ref, qej_ref, out_ref):
    i = pl.program_id(0)
    j = pl.program_id(1)

    @pl.when(j >= i)
    def _():
        t = lax.dot_general(qei_ref[...], qej_ref[...],
                            (((1,), (1,)), ((), ())),
                            preferred_element_type=jnp.float32)
        tmn = jnp.min(t)
        tmx = jnp.max(t)
        first = jnp.logical_and(i == 0, j == 0)

        @pl.when(first)
        def _():
            out_ref[0] = tmn
            out_ref[1] = tmx

        @pl.when(jnp.logical_not(first))
        def _():
            out_ref[0] = jnp.minimum(out_ref[0], tmn)
            out_ref[1] = jnp.maximum(out_ref[1], tmx)


def _minmax_pallas(q_edge):
    return pl.pallas_call(
        _minmax_body,
        grid=(N // BQ, N // BQ),
        in_specs=[
            pl.BlockSpec((BQ, D), lambda i, j: (i, 0)),
            pl.BlockSpec((BQ, D), lambda i, j: (j, 0)),
        ],
        out_specs=pl.BlockSpec(memory_space=pltpu.SMEM),
        out_shape=jax.ShapeDtypeStruct((2,), jnp.float32),
    )(q_edge, q_edge)


def _gcn2_body(adj_ref, x_ref, degp_ref, w_ref, b_ref, wl_ref, bl_ref,
               h2_ref, o_ref, c_ref):
    i = pl.program_id(0)
    u = jnp.dot(adj_ref[...], x_ref[...], preferred_element_type=jnp.float32)
    cpart = jnp.sum(x_ref[pl.ds(i * BI, BI), :] * u)

    @pl.when(i == 0)
    def _():
        c_ref[0] = cpart

    @pl.when(i > 0)
    def _():
        c_ref[0] = c_ref[0] + cpart

    deg_out = degp_ref[0, 1, :] + degp_ref[1, 1, :]
    nsrc = lax.rsqrt(jnp.clip(deg_out, 1.0, None))
    deg_in = (degp_ref[0, 0, pl.ds(i * BI, BI)] +
              degp_ref[1, 0, pl.ds(i * BI, BI)])
    ndst = lax.rsqrt(jnp.clip(deg_in, 1.0, None))
    xs = x_ref[...] * nsrc[:, None]
    agg = _exact_agg(adj_ref[...], xs)
    agg = agg * ndst[:, None]
    h = jnp.dot(agg, w_ref[...], preferred_element_type=jnp.float32) + b_ref[...]
    h2 = jnp.maximum(h, 0.0)
    h2_ref[...] = h2
    o_ref[...] = lax.dot_general(h2, wl_ref[...], (((1,), (1,)), ((), ())),
                                 preferred_element_type=jnp.float32) + bl_ref[...]


def _gcn2_pallas(adj, q_edge, degp, w2, b2, wl, bl):
    return pl.pallas_call(
        _gcn2_body,
        grid=(N // BI,),
        in_specs=[
            pl.BlockSpec((BI, N), lambda i: (i, 0)),
            pl.BlockSpec((N, D), lambda i: (0, 0)),
            pl.BlockSpec((2, 2, N), lambda i: (0, 0, 0)),
            pl.BlockSpec((D, D), lambda i: (0, 0)),
            pl.BlockSpec((1, D), lambda i: (0, 0)),
            pl.BlockSpec((DOUT, D), lambda i: (0, 0)),
            pl.BlockSpec((1, DOUT), lambda i: (0, 0)),
        ],
        out_specs=[
            pl.BlockSpec((BI, D), lambda i: (i, 0)),
            pl.BlockSpec((BI, DOUT), lambda i: (i, 0)),
            pl.BlockSpec(memory_space=pltpu.SMEM),
        ],
        out_shape=[
            jax.ShapeDtypeStruct((N, D), jnp.float32),
            jax.ShapeDtypeStruct((N, DOUT), jnp.float32),
            jax.ShapeDtypeStruct((1,), jnp.float32),
        ],
    )(adj, q_edge, degp, w2, b2, wl, bl)


def kernel(feats, edge_index, W1, b1, codebook, Wd1, bd1, Wd2, bd2,
           W2, b2, Wl, bl):
    src = edge_index[0]
    dst = edge_index[1]

    adj_flat, degp = _get_adj_build()(src, dst)
    adj = adj_flat.reshape(N, N)
    dup = _get_dup_count()(adj_flat, src, dst)

    h1 = _gcn_layer_pallas(adj, feats, degp, W1, b1.reshape(1, D))
    cn = _l2norm_pallas(codebook)
    dist, ind2 = _vq_pallas(h1, cn)
    ind = ind2.reshape(N)
    quant = _get_quant_gather()(cn, ind)

    q_edge, sse_commit, sse_node, gr = _dec_pallas(
        quant, h1, Wd1, bd1.reshape(1, D), Wd2, bd2.reshape(1, D))

    mm = _minmax_pallas(q_edge)
    mn, mx, s1, s2 = mm[0], mm[1], gr[0], gr[1]
    a2 = jnp.sum(dup)
    a1 = jnp.float32(E)

    h2, out, cvec = _gcn2_pallas(adj, q_edge, degp, W2, b2.reshape(1, D),
                                 Wl, bl.reshape(1, DOUT))
    c = cvec[0]

    nn = jnp.float32(N) * jnp.float32(N)
    den = mx - mn
    s2n = (s2 - 2.0 * mn * s1 + nn * mn * mn) / (den * den)
    cxn = (c - mn * a1) / den
    edge_rec = jnp.sqrt((a2 - 2.0 * cxn + s2n) / nn)
    feature_rec = sse_node[0, 0] / jnp.float32(N * D)
    commit = 0.25 * sse_commit[0, 0] / jnp.float32(N * D)
    loss = feature_rec + edge_rec + commit

    return (h1, quant, h2, out, loss, dist, cn)
